# Initial kernel scaffold; baseline (speedup 1.0000x reference)
import jax
import jax.numpy as jnp
from jax.experimental import pallas as pl


def kernel(questions, e_s, answers, subj_idx, rel_idx, obj_idx, W_step0, b_step0, W_step1, b_step1, W_cq, b_cq, rel_emb, ent_emb, ent_bias):
    raise NotImplementedError("write your pallas kernel here")


# speedup vs baseline: 11.1195x; 11.1195x over previous
import jax
import jax.numpy as jnp
from jax import lax
from jax.experimental import pallas as pl
from jax.experimental.pallas import tpu as pltpu
from jax.experimental.pallas import tpu_sc as plsc

E = 50000
R = 500
D = 128
B = 32
T = 800000

NC, NS = 2, 16
NW = NC * NS
SUB = 128
K = 2
CHUNK = K * SUB
RPAD = 512
TP = 802816
TW = TP // NW
NCHUNK = TW // CHUNK
IDXC = 7
NIDX = NCHUNK // IDXC
EPC = E // NS

EP = 50176
BLK = 2000
NBLK = E // BLK
BLK2 = EP // 8
NBLK2 = 8


def _follow_body(eT, rdT, subj2, rel2, obj2, out,
                 acc, idx_s, idx_r, idx_o, rows_e, rows_r, sem_e, sem_r):
    c = lax.axis_index("c")
    s = lax.axis_index("s")
    w = s * NC + c
    base = s * EPC

    def zrow(i, carry):
        rows_e[i, pl.ds(0, 16)] = jnp.zeros((16,), jnp.float32)
        rows_e[i, pl.ds(16, 16)] = jnp.zeros((16,), jnp.float32)
        return carry
    lax.fori_loop(0, CHUNK, zrow, 0)
    for k in range(EPC // CHUNK):
        pltpu.sync_copy(rows_e.at[pl.ds(0, CHUNK)],
                        acc.at[pl.ds(base + k * CHUNK, CHUNK)])
    rem = EPC % CHUNK
    if rem:
        pltpu.sync_copy(rows_e.at[pl.ds(0, rem)],
                        acc.at[pl.ds(base + (EPC // CHUNK) * CHUNK, rem)])
    plsc.subcore_barrier()

    def stage_body(ib, carry):
        r0 = w * (NCHUNK * K) + ib * (IDXC * K)
        pltpu.sync_copy(subj2.at[pl.ds(r0, IDXC * K)], idx_s)
        pltpu.sync_copy(rel2.at[pl.ds(r0, IDXC * K)], idx_r)
        pltpu.sync_copy(obj2.at[pl.ds(r0, IDXC * K)], idx_o)

        def chunk_body(ic, ccarry):
            j0 = ic * K
            cps = []
            for j in range(K):
                cps.append(pltpu.async_copy(
                    eT.at[idx_s.at[j0 + j]],
                    rows_e.at[pl.ds(j * SUB, SUB)], sem_e))
                cps.append(pltpu.async_copy(
                    rdT.at[idx_r.at[j0 + j]],
                    rows_r.at[pl.ds(j * SUB, SUB)], sem_r))
            for cp in cps:
                cp.wait()

            def mul_body(e, mcarry):
                rows_e[e, pl.ds(0, 16)] = (rows_e[e, pl.ds(0, 16)] *
                                           rows_r[e, pl.ds(0, 16)])
                rows_e[e, pl.ds(16, 16)] = (rows_e[e, pl.ds(16, 16)] *
                                            rows_r[e, pl.ds(16, 16)])
                return mcarry
            lax.fori_loop(0, CHUNK, mul_body, 0)

            for j in range(K):
                pltpu.sync_copy(rows_e.at[pl.ds(j * SUB, SUB)],
                                acc.at[idx_o.at[j0 + j]], add=True)
            return ccarry
        lax.fori_loop(0, IDXC, chunk_body, 0)
        return carry
    lax.fori_loop(0, NIDX, stage_body, 0)

    plsc.subcore_barrier()
    pltpu.sync_copy(acc.at[pl.ds(base, EPC)],
                    out.at[pl.ds(c * E + base, EPC)])


def _follow_sc(eT, rdT, subj2, rel2, obj2):
    f = pl.kernel(
        _follow_body,
        out_type=jax.ShapeDtypeStruct((NC * E, B), jnp.float32),
        mesh=plsc.VectorSubcoreMesh(core_axis_name="c", subcore_axis_name="s"),
        scratch_types=[
            pltpu.VMEM_SHARED((E, B), jnp.float32),
            pltpu.VMEM((IDXC * K, SUB), jnp.int32),
            pltpu.VMEM((IDXC * K, SUB), jnp.int32),
            pltpu.VMEM((IDXC * K, SUB), jnp.int32),
            pltpu.VMEM((CHUNK, B), jnp.float32),
            pltpu.VMEM((CHUNK, B), jnp.float32),
            pltpu.SemaphoreType.DMA,
            pltpu.SemaphoreType.DMA,
        ],
        compiler_params=pltpu.CompilerParams(use_tc_tiling_on_sc=False),
    )
    return f(eT, rdT, subj2, rel2, obj2)


def _prep_body(qs, W0, b0, W1, b1, re_, rdT0_o, rdT1_o, gt_o, hs_o):
    relv = re_[...]
    rows = lax.broadcasted_iota(jnp.int32, (R, B), 0)
    qcol = qs[...][:, 0]
    qm = rows == qcol[None, :]
    q_emb = lax.dot_general(jnp.where(qm, 1.0, 0.0), relv,
                            (((0,), (0,)), ((), ())),
                            preferred_element_type=jnp.float32)

    def rel_dist_T(W, b):
        cq = jnp.tanh(jnp.dot(q_emb, W[...],
                              preferred_element_type=jnp.float32) + b[...])
        lgT = lax.dot_general(relv, cq, (((1,), (1,)), ((), ())),
                              preferred_element_type=jnp.float32)
        mx = jnp.max(lgT, axis=0, keepdims=True)
        ex = jnp.exp(lgT - mx)
        return ex / jnp.sum(ex, axis=0, keepdims=True)

    rdT0 = rel_dist_T(W0, b0)
    rdT1 = rel_dist_T(W1, b1)
    zpad = jnp.zeros((RPAD - R, B), jnp.float32)
    rdT0_o[...] = jnp.concatenate([rdT0, zpad], axis=0)
    rdT1_o[...] = jnp.concatenate([rdT1, zpad], axis=0)

    gt_o[...] = jnp.sum(jnp.where(qm, rdT0, 0.0), axis=0).reshape(1, B)

    def argmax0(rdT):
        amax = jnp.max(rdT, axis=0, keepdims=True)
        return jnp.min(jnp.where(rdT == amax, rows, R), axis=0)

    r0 = argmax0(rdT0)
    r1 = argmax0(rdT1)
    cond2 = (jnp.abs(r0 - r1) == 1) & (jnp.minimum(r0, r1) % 2 == 0)
    nself = (r0 == 0).astype(jnp.int32) + (r1 == 0).astype(jnp.int32)
    cond3 = nself == 1
    hs = (1.0 - cond2.astype(jnp.float32)) * (1.0 - cond3.astype(jnp.float32))
    hs_o[...] = hs.reshape(1, B)


def _prep(questions, W0, b0, W1, b1, rel_emb):
    return pl.pallas_call(
        _prep_body,
        out_shape=[
            jax.ShapeDtypeStruct((RPAD, B), jnp.float32),
            jax.ShapeDtypeStruct((RPAD, B), jnp.float32),
            jax.ShapeDtypeStruct((1, B), jnp.float32),
            jax.ShapeDtypeStruct((1, B), jnp.float32),
        ],
    )(questions, W0, b0, W1, b1, rel_emb)


def _combine0_body(gt, p0, p1, ansT, out):
    x = p0[0] + p1[0] - ansT[...] * gt[...]
    out[...] = jnp.where(x > 1.0, 1.0, x)


def _combine0(gt, P3, ansT):
    return pl.pallas_call(
        _combine0_body,
        grid=(NBLK,),
        in_specs=[
            pl.BlockSpec((1, B), lambda i: (0, 0)),
            pl.BlockSpec((1, BLK, B), lambda i: (0, i, 0)),
            pl.BlockSpec((1, BLK, B), lambda i: (1, i, 0)),
            pl.BlockSpec((BLK, B), lambda i: (i, 0)),
        ],
        out_specs=pl.BlockSpec((BLK, B), lambda i: (i, 0)),
        out_shape=jax.ShapeDtypeStruct((E, B), jnp.float32),
    )(gt, P3, P3, ansT)


def _pass1_body(hs, p0, p1, esT, ansT, emb, UT_o, s_o, q2_o, ea_o):
    i = pl.program_id(0)
    x = p0[0] + p1[0]
    x = jnp.where(x > 1.0, 1.0, x)
    x = x * (1.0 - esT[...] * (1.0 - hs[...]))

    @pl.when(i == 0)
    def _():
        UT_o[...] = jnp.zeros((D, B), jnp.float32)
        s_o[...] = jnp.zeros((1, B), jnp.float32)
        q2_o[...] = jnp.zeros((1, B), jnp.float32)
        ea_o[...] = jnp.zeros((1, B), jnp.float32)

    UT_o[...] += lax.dot_general(emb[...], x, (((0,), (0,)), ((), ())),
                                 preferred_element_type=jnp.float32)
    s_o[...] += jnp.sum(x, axis=0, keepdims=True)
    q2_o[...] += jnp.sum(x * x, axis=0, keepdims=True)
    ea_o[...] += jnp.sum(ansT[...] * x, axis=0, keepdims=True)


def _pass1(hs, P3, esT, ansT, ent_emb):
    return pl.pallas_call(
        _pass1_body,
        grid=(NBLK,),
        in_specs=[
            pl.BlockSpec((1, B), lambda i: (0, 0)),
            pl.BlockSpec((1, BLK, B), lambda i: (0, i, 0)),
            pl.BlockSpec((1, BLK, B), lambda i: (1, i, 0)),
            pl.BlockSpec((BLK, B), lambda i: (i, 0)),
            pl.BlockSpec((BLK, B), lambda i: (i, 0)),
            pl.BlockSpec((BLK, D), lambda i: (i, 0)),
        ],
        out_specs=[
            pl.BlockSpec((D, B), lambda i: (0, 0)),
            pl.BlockSpec((1, B), lambda i: (0, 0)),
            pl.BlockSpec((1, B), lambda i: (0, 0)),
            pl.BlockSpec((1, B), lambda i: (0, 0)),
        ],
        out_shape=[
            jax.ShapeDtypeStruct((D, B), jnp.float32),
            jax.ShapeDtypeStruct((1, B), jnp.float32),
            jax.ShapeDtypeStruct((1, B), jnp.float32),
            jax.ShapeDtypeStruct((1, B), jnp.float32),
        ],
    )(hs, P3, P3, esT, ansT, ent_emb)


def _pass2_body(UT, s, q2, ea, emb, ansT, bias,
                mr_o, sr_o, pa_o, ls_o, lp_o):
    i = pl.program_id(0)
    pmT = UT[...] / (s[...] + 1e-6)
    P = lax.dot_general(emb[...], pmT, (((1,), (0,)), ((), ())),
                        preferred_element_type=jnp.float32) + bias[...]
    bm = jnp.max(P, axis=0, keepdims=True)
    bpa = jnp.sum(ansT[...] * P, axis=0, keepdims=True)

    @pl.when(i == 0)
    def _():
        mr_o[...] = bm
        sr_o[...] = jnp.sum(jnp.exp(P - bm), axis=0, keepdims=True)
        pa_o[...] = bpa

    @pl.when(i > 0)
    def _():
        nm = jnp.maximum(mr_o[...], bm)
        sr_o[...] = (sr_o[...] * jnp.exp(mr_o[...] - nm) +
                     jnp.sum(jnp.exp(P - nm), axis=0, keepdims=True))
        mr_o[...] = nm
        pa_o[...] += bpa

    @pl.when(i == NBLK2 - 1)
    def _():
        lse = jnp.log(sr_o[...]) + mr_o[...]
        lp_o[...] = (-jnp.sum(pa_o[...] - lse) / B).reshape(1, 1)
        eav = ea[...]
        ls_o[...] = ((jnp.sum(q2[...]) - jnp.sum(eav * eav) +
                      jnp.sum(10.0 * (eav - 1.0) ** 2)) / (E * B)).reshape(1, 1)


def _pass2(UT, s, q2, ea, embP, ansTP, biasP):
    return pl.pallas_call(
        _pass2_body,
        grid=(NBLK2,),
        in_specs=[
            pl.BlockSpec((D, B), lambda i: (0, 0)),
            pl.BlockSpec((1, B), lambda i: (0, 0)),
            pl.BlockSpec((1, B), lambda i: (0, 0)),
            pl.BlockSpec((1, B), lambda i: (0, 0)),
            pl.BlockSpec((BLK2, D), lambda i: (i, 0)),
            pl.BlockSpec((BLK2, B), lambda i: (i, 0)),
            pl.BlockSpec((BLK2, 1), lambda i: (i, 0)),
        ],
        out_specs=[
            pl.BlockSpec((1, B), lambda i: (0, 0)),
            pl.BlockSpec((1, B), lambda i: (0, 0)),
            pl.BlockSpec((1, B), lambda i: (0, 0)),
            pl.BlockSpec((1, 1), lambda i: (0, 0)),
            pl.BlockSpec((1, 1), lambda i: (0, 0)),
        ],
        out_shape=[
            jax.ShapeDtypeStruct((1, B), jnp.float32),
            jax.ShapeDtypeStruct((1, B), jnp.float32),
            jax.ShapeDtypeStruct((1, B), jnp.float32),
            jax.ShapeDtypeStruct((1, 1), jnp.float32),
            jax.ShapeDtypeStruct((1, 1), jnp.float32),
        ],
    )(UT, s, q2, ea, embP, ansTP, biasP)


def kernel(questions, e_s, answers, subj_idx, rel_idx, obj_idx,
           W_step0, b_step0, W_step1, b_step1, W_cq, b_cq,
           rel_emb, ent_emb, ent_bias):
    eT0 = e_s.T
    ansT = answers.T
    npad = TP - T
    subj2 = jnp.concatenate(
        [subj_idx, jnp.zeros((npad,), jnp.int32)]).reshape(TP // SUB, SUB)
    rel2 = jnp.concatenate(
        [rel_idx, jnp.full((npad,), R, jnp.int32)]).reshape(TP // SUB, SUB)
    obj2 = jnp.concatenate(
        [obj_idx, jnp.zeros((npad,), jnp.int32)]).reshape(TP // SUB, SUB)

    rdT0, rdT1, gt, hs = _prep(questions, W_step0, b_step0,
                               W_step1, b_step1, rel_emb)

    P0 = _follow_sc(eT0, rdT0, subj2, rel2, obj2).reshape(NC, E, B)
    e1T = _combine0(gt, P0, ansT)
    P1 = _follow_sc(e1T, rdT1, subj2, rel2, obj2).reshape(NC, E, B)

    UT, s, q2, ea = _pass1(hs, P1, eT0, ansT, ent_emb)

    embP = jnp.pad(ent_emb, ((0, EP - E), (0, 0)))
    ansTP = jnp.pad(ansT, ((0, EP - E), (0, 0)))
    biasP = jnp.pad(ent_bias, (0, EP - E),
                    constant_values=-1e30).reshape(EP, 1)
    _, _, _, ls, lp = _pass2(UT, s, q2, ea, embP, ansTP, biasP)
    return (ls[0, 0], lp[0, 0])


# speedup vs baseline: 14.4061x; 1.2956x over previous
import jax
import jax.numpy as jnp
from jax import lax
from jax.experimental import pallas as pl
from jax.experimental.pallas import tpu as pltpu
from jax.experimental.pallas import tpu_sc as plsc

E = 50000
R = 500
D = 128
B = 32
T = 800000

NC, NS = 2, 16
NW = NC * NS
SUB = 128
RPAD = 512
TP = 802816
TW = TP // NW
NG = TW // SUB
GSTAGE = 14
NSTAGE = NG // GSTAGE
CHUNK = 2 * SUB
EPC = E // NS

EP = 50176
BLK = 2000
NBLK = E // BLK
BLK2 = EP // 8
NBLK2 = 8


def _follow_body(eT, rdT, subj2, rel2, obj2, out,
                 acc, idx_s, idx_r, idx_o, rows_e, rows_r,
                 sem_e, sem_r, sem_i, sem_w):
    c = lax.axis_index("c")
    s = lax.axis_index("s")
    w = s * NC + c
    base = s * EPC

    def zrow(i, carry):
        rows_e[i, pl.ds(0, 16)] = jnp.zeros((16,), jnp.float32)
        rows_e[i, pl.ds(16, 16)] = jnp.zeros((16,), jnp.float32)
        return carry
    lax.fori_loop(0, CHUNK, zrow, 0)
    for k in range(EPC // CHUNK):
        pltpu.sync_copy(rows_e.at[pl.ds(0, CHUNK)],
                        acc.at[pl.ds(base + k * CHUNK, CHUNK)])
    rem = EPC % CHUNK
    if rem:
        pltpu.sync_copy(rows_e.at[pl.ds(0, rem)],
                        acc.at[pl.ds(base + (EPC // CHUNK) * CHUNK, rem)])
    plsc.subcore_barrier()

    def stage_body(st, carry):
        bufbase = lax.rem(st, 2) * GSTAGE

        @pl.when(st == 0)
        def _():
            r0 = w * NG
            pltpu.async_copy(subj2.at[pl.ds(r0, GSTAGE)],
                             idx_s.at[pl.ds(0, GSTAGE)], sem_i)
            pltpu.async_copy(rel2.at[pl.ds(r0, GSTAGE)],
                             idx_r.at[pl.ds(0, GSTAGE)], sem_i)
            pltpu.async_copy(obj2.at[pl.ds(r0, GSTAGE)],
                             idx_o.at[pl.ds(0, GSTAGE)], sem_i)
        for _ in range(3):
            pltpu.make_async_copy(subj2.at[pl.ds(0, GSTAGE)],
                                  idx_s.at[pl.ds(0, GSTAGE)], sem_i).wait()

        @pl.when(st + 1 < NSTAGE)
        def _():
            r1 = w * NG + (st + 1) * GSTAGE
            nbase = (GSTAGE - bufbase)
            pltpu.async_copy(subj2.at[pl.ds(r1, GSTAGE)],
                             idx_s.at[pl.ds(nbase, GSTAGE)], sem_i)
            pltpu.async_copy(rel2.at[pl.ds(r1, GSTAGE)],
                             idx_r.at[pl.ds(nbase, GSTAGE)], sem_i)
            pltpu.async_copy(obj2.at[pl.ds(r1, GSTAGE)],
                             idx_o.at[pl.ds(nbase, GSTAGE)], sem_i)

        ge, gr, sc = {}, {}, {}
        ge[0] = pltpu.async_copy(eT.at[idx_s.at[bufbase]],
                                 rows_e.at[pl.ds(0, SUB)], sem_e)
        gr[0] = pltpu.async_copy(rdT.at[idx_r.at[bufbase]],
                                 rows_r.at[pl.ds(0, SUB)], sem_r)
        for p in range(GSTAGE):
            slot = p % 2
            if p + 1 < GSTAGE:
                if p >= 1:
                    sc[p - 1].wait()
                off = (1 - slot) * SUB
                ge[p + 1] = pltpu.async_copy(
                    eT.at[idx_s.at[bufbase + p + 1]],
                    rows_e.at[pl.ds(off, SUB)], sem_e)
                gr[p + 1] = pltpu.async_copy(
                    rdT.at[idx_r.at[bufbase + p + 1]],
                    rows_r.at[pl.ds(off, SUB)], sem_r)
            ge[p].wait()
            gr[p].wait()

            soff = slot * SUB

            @plsc.parallel_loop(0, SUB, 1, unroll=4)
            def _(r):
                rows_e[soff + r, pl.ds(0, 16)] = (
                    rows_e[soff + r, pl.ds(0, 16)] *
                    rows_r[soff + r, pl.ds(0, 16)])
                rows_e[soff + r, pl.ds(16, 16)] = (
                    rows_e[soff + r, pl.ds(16, 16)] *
                    rows_r[soff + r, pl.ds(16, 16)])

            sc[p] = pltpu.async_copy(rows_e.at[pl.ds(soff, SUB)],
                                     acc.at[idx_o.at[bufbase + p]],
                                     sem_w, add=True)
        sc[GSTAGE - 2].wait()
        sc[GSTAGE - 1].wait()
        return carry
    lax.fori_loop(0, NSTAGE, stage_body, 0)

    plsc.subcore_barrier()
    pltpu.sync_copy(acc.at[pl.ds(base, EPC)],
                    out.at[pl.ds(c * E + base, EPC)])


def _follow_sc(eT, rdT, subj2, rel2, obj2):
    f = pl.kernel(
        _follow_body,
        out_type=jax.ShapeDtypeStruct((NC * E, B), jnp.float32),
        mesh=plsc.VectorSubcoreMesh(core_axis_name="c", subcore_axis_name="s"),
        scratch_types=[
            pltpu.VMEM_SHARED((E, B), jnp.float32),
            pltpu.VMEM((2 * GSTAGE, SUB), jnp.int32),
            pltpu.VMEM((2 * GSTAGE, SUB), jnp.int32),
            pltpu.VMEM((2 * GSTAGE, SUB), jnp.int32),
            pltpu.VMEM((CHUNK, B), jnp.float32),
            pltpu.VMEM((CHUNK, B), jnp.float32),
            pltpu.SemaphoreType.DMA,
            pltpu.SemaphoreType.DMA,
            pltpu.SemaphoreType.DMA,
            pltpu.SemaphoreType.DMA,
        ],
        compiler_params=pltpu.CompilerParams(use_tc_tiling_on_sc=False),
    )
    return f(eT, rdT, subj2, rel2, obj2)


def _prep_body(qs, W0, b0, W1, b1, re_, rdT0_o, rdT1_o, gt_o, hs_o):
    relv = re_[...]
    rows = lax.broadcasted_iota(jnp.int32, (R, B), 0)
    qcol = qs[...][:, 0]
    qm = rows == qcol[None, :]
    q_emb = lax.dot_general(jnp.where(qm, 1.0, 0.0), relv,
                            (((0,), (0,)), ((), ())),
                            preferred_element_type=jnp.float32)

    def rel_dist_T(W, b):
        cq = jnp.tanh(jnp.dot(q_emb, W[...],
                              preferred_element_type=jnp.float32) + b[...])
        lgT = lax.dot_general(relv, cq, (((1,), (1,)), ((), ())),
                              preferred_element_type=jnp.float32)
        mx = jnp.max(lgT, axis=0, keepdims=True)
        ex = jnp.exp(lgT - mx)
        return ex / jnp.sum(ex, axis=0, keepdims=True)

    rdT0 = rel_dist_T(W0, b0)
    rdT1 = rel_dist_T(W1, b1)
    zpad = jnp.zeros((RPAD - R, B), jnp.float32)
    rdT0_o[...] = jnp.concatenate([rdT0, zpad], axis=0)
    rdT1_o[...] = jnp.concatenate([rdT1, zpad], axis=0)

    gt_o[...] = jnp.sum(jnp.where(qm, rdT0, 0.0), axis=0).reshape(1, B)

    def argmax0(rdT):
        amax = jnp.max(rdT, axis=0, keepdims=True)
        return jnp.min(jnp.where(rdT == amax, rows, R), axis=0)

    r0 = argmax0(rdT0)
    r1 = argmax0(rdT1)
    cond2 = (jnp.abs(r0 - r1) == 1) & (jnp.minimum(r0, r1) % 2 == 0)
    nself = (r0 == 0).astype(jnp.int32) + (r1 == 0).astype(jnp.int32)
    cond3 = nself == 1
    hs = (1.0 - cond2.astype(jnp.float32)) * (1.0 - cond3.astype(jnp.float32))
    hs_o[...] = hs.reshape(1, B)


def _prep(questions, W0, b0, W1, b1, rel_emb):
    return pl.pallas_call(
        _prep_body,
        out_shape=[
            jax.ShapeDtypeStruct((RPAD, B), jnp.float32),
            jax.ShapeDtypeStruct((RPAD, B), jnp.float32),
            jax.ShapeDtypeStruct((1, B), jnp.float32),
            jax.ShapeDtypeStruct((1, B), jnp.float32),
        ],
    )(questions, W0, b0, W1, b1, rel_emb)


def _combine0_body(gt, p0, p1, ansT, out):
    x = p0[0] + p1[0] - ansT[...] * gt[...]
    out[...] = jnp.where(x > 1.0, 1.0, x)


def _combine0(gt, P3, ansT):
    return pl.pallas_call(
        _combine0_body,
        grid=(NBLK,),
        in_specs=[
            pl.BlockSpec((1, B), lambda i: (0, 0)),
            pl.BlockSpec((1, BLK, B), lambda i: (0, i, 0)),
            pl.BlockSpec((1, BLK, B), lambda i: (1, i, 0)),
            pl.BlockSpec((BLK, B), lambda i: (i, 0)),
        ],
        out_specs=pl.BlockSpec((BLK, B), lambda i: (i, 0)),
        out_shape=jax.ShapeDtypeStruct((E, B), jnp.float32),
    )(gt, P3, P3, ansT)


def _pass1_body(hs, p0, p1, esT, ansT, emb, UT_o, s_o, q2_o, ea_o):
    i = pl.program_id(0)
    x = p0[0] + p1[0]
    x = jnp.where(x > 1.0, 1.0, x)
    x = x * (1.0 - esT[...] * (1.0 - hs[...]))

    @pl.when(i == 0)
    def _():
        UT_o[...] = jnp.zeros((D, B), jnp.float32)
        s_o[...] = jnp.zeros((1, B), jnp.float32)
        q2_o[...] = jnp.zeros((1, B), jnp.float32)
        ea_o[...] = jnp.zeros((1, B), jnp.float32)

    UT_o[...] += lax.dot_general(emb[...], x, (((0,), (0,)), ((), ())),
                                 preferred_element_type=jnp.float32)
    s_o[...] += jnp.sum(x, axis=0, keepdims=True)
    q2_o[...] += jnp.sum(x * x, axis=0, keepdims=True)
    ea_o[...] += jnp.sum(ansT[...] * x, axis=0, keepdims=True)


def _pass1(hs, P3, esT, ansT, ent_emb):
    return pl.pallas_call(
        _pass1_body,
        grid=(NBLK,),
        in_specs=[
            pl.BlockSpec((1, B), lambda i: (0, 0)),
            pl.BlockSpec((1, BLK, B), lambda i: (0, i, 0)),
            pl.BlockSpec((1, BLK, B), lambda i: (1, i, 0)),
            pl.BlockSpec((BLK, B), lambda i: (i, 0)),
            pl.BlockSpec((BLK, B), lambda i: (i, 0)),
            pl.BlockSpec((BLK, D), lambda i: (i, 0)),
        ],
        out_specs=[
            pl.BlockSpec((D, B), lambda i: (0, 0)),
            pl.BlockSpec((1, B), lambda i: (0, 0)),
            pl.BlockSpec((1, B), lambda i: (0, 0)),
            pl.BlockSpec((1, B), lambda i: (0, 0)),
        ],
        out_shape=[
            jax.ShapeDtypeStruct((D, B), jnp.float32),
            jax.ShapeDtypeStruct((1, B), jnp.float32),
            jax.ShapeDtypeStruct((1, B), jnp.float32),
            jax.ShapeDtypeStruct((1, B), jnp.float32),
        ],
    )(hs, P3, P3, esT, ansT, ent_emb)


def _pass2_body(UT, s, q2, ea, emb, ansT, bias,
                mr_o, sr_o, pa_o, ls_o, lp_o):
    i = pl.program_id(0)
    pmT = UT[...] / (s[...] + 1e-6)
    P = lax.dot_general(emb[...], pmT, (((1,), (0,)), ((), ())),
                        preferred_element_type=jnp.float32) + bias[...]
    bm = jnp.max(P, axis=0, keepdims=True)
    bpa = jnp.sum(ansT[...] * P, axis=0, keepdims=True)

    @pl.when(i == 0)
    def _():
        mr_o[...] = bm
        sr_o[...] = jnp.sum(jnp.exp(P - bm), axis=0, keepdims=True)
        pa_o[...] = bpa

    @pl.when(i > 0)
    def _():
        nm = jnp.maximum(mr_o[...], bm)
        sr_o[...] = (sr_o[...] * jnp.exp(mr_o[...] - nm) +
                     jnp.sum(jnp.exp(P - nm), axis=0, keepdims=True))
        mr_o[...] = nm
        pa_o[...] += bpa

    @pl.when(i == NBLK2 - 1)
    def _():
        lse = jnp.log(sr_o[...]) + mr_o[...]
        lp_o[...] = (-jnp.sum(pa_o[...] - lse) / B).reshape(1, 1)
        eav = ea[...]
        ls_o[...] = ((jnp.sum(q2[...]) - jnp.sum(eav * eav) +
                      jnp.sum(10.0 * (eav - 1.0) ** 2)) / (E * B)).reshape(1, 1)


def _pass2(UT, s, q2, ea, embP, ansTP, biasP):
    return pl.pallas_call(
        _pass2_body,
        grid=(NBLK2,),
        in_specs=[
            pl.BlockSpec((D, B), lambda i: (0, 0)),
            pl.BlockSpec((1, B), lambda i: (0, 0)),
            pl.BlockSpec((1, B), lambda i: (0, 0)),
            pl.BlockSpec((1, B), lambda i: (0, 0)),
            pl.BlockSpec((BLK2, D), lambda i: (i, 0)),
            pl.BlockSpec((BLK2, B), lambda i: (i, 0)),
            pl.BlockSpec((BLK2, 1), lambda i: (i, 0)),
        ],
        out_specs=[
            pl.BlockSpec((1, B), lambda i: (0, 0)),
            pl.BlockSpec((1, B), lambda i: (0, 0)),
            pl.BlockSpec((1, B), lambda i: (0, 0)),
            pl.BlockSpec((1, 1), lambda i: (0, 0)),
            pl.BlockSpec((1, 1), lambda i: (0, 0)),
        ],
        out_shape=[
            jax.ShapeDtypeStruct((1, B), jnp.float32),
            jax.ShapeDtypeStruct((1, B), jnp.float32),
            jax.ShapeDtypeStruct((1, B), jnp.float32),
            jax.ShapeDtypeStruct((1, 1), jnp.float32),
            jax.ShapeDtypeStruct((1, 1), jnp.float32),
        ],
    )(UT, s, q2, ea, embP, ansTP, biasP)


def kernel(questions, e_s, answers, subj_idx, rel_idx, obj_idx,
           W_step0, b_step0, W_step1, b_step1, W_cq, b_cq,
           rel_emb, ent_emb, ent_bias):
    eT0 = e_s.T
    ansT = answers.T
    npad = TP - T
    subj2 = jnp.concatenate(
        [subj_idx, jnp.zeros((npad,), jnp.int32)]).reshape(TP // SUB, SUB)
    rel2 = jnp.concatenate(
        [rel_idx, jnp.full((npad,), R, jnp.int32)]).reshape(TP // SUB, SUB)
    obj2 = jnp.concatenate(
        [obj_idx, jnp.zeros((npad,), jnp.int32)]).reshape(TP // SUB, SUB)

    rdT0, rdT1, gt, hs = _prep(questions, W_step0, b_step0,
                               W_step1, b_step1, rel_emb)

    P0 = _follow_sc(eT0, rdT0, subj2, rel2, obj2).reshape(NC, E, B)
    e1T = _combine0(gt, P0, ansT)
    P1 = _follow_sc(e1T, rdT1, subj2, rel2, obj2).reshape(NC, E, B)

    UT, s, q2, ea = _pass1(hs, P1, eT0, ansT, ent_emb)

    embP = jnp.pad(ent_emb, ((0, EP - E), (0, 0)))
    ansTP = jnp.pad(ansT, ((0, EP - E), (0, 0)))
    biasP = jnp.pad(ent_bias, (0, EP - E),
                    constant_values=-1e30).reshape(EP, 1)
    _, _, _, ls, lp = _pass2(UT, s, q2, ea, embP, ansTP, biasP)
    return (ls[0, 0], lp[0, 0])


# speedup vs baseline: 16.7329x; 1.1615x over previous
import jax
import jax.numpy as jnp
from jax import lax
from jax.experimental import pallas as pl
from jax.experimental.pallas import tpu as pltpu
from jax.experimental.pallas import tpu_sc as plsc

E = 50000
R = 500
D = 128
B = 32
T = 800000

NC, NS = 2, 16
NW = NC * NS
SUB = 128
RPAD = 512
TP = 802816
TW = TP // NW
NG = TW // SUB
GSTAGE = 14
NSTAGE = NG // GSTAGE
CHUNK = 2 * SUB
EPC = E // NS

EP = 50176
BLK = 2000
NBLK = E // BLK
BLK2 = EP // 8
NBLK2 = 8


def _follow_body(eT, rdT, subj2, rel2, obj2, out,
                 acc, rd_sh, idx_s, idx_r, idx_o, rows_e, rows_r,
                 sem_e, sem_r, sem_i, sem_w):
    c = lax.axis_index("c")
    s = lax.axis_index("s")
    w = s * NC + c
    base = s * EPC

    @pl.when(s == 0)
    def _():
        pltpu.sync_copy(rdT, rd_sh)

    def zrow(i, carry):
        rows_e[i, pl.ds(0, 16)] = jnp.zeros((16,), jnp.float32)
        rows_e[i, pl.ds(16, 16)] = jnp.zeros((16,), jnp.float32)
        return carry
    lax.fori_loop(0, CHUNK, zrow, 0)
    for k in range(EPC // CHUNK):
        pltpu.sync_copy(rows_e.at[pl.ds(0, CHUNK)],
                        acc.at[pl.ds(base + k * CHUNK, CHUNK)])
    rem = EPC % CHUNK
    if rem:
        pltpu.sync_copy(rows_e.at[pl.ds(0, rem)],
                        acc.at[pl.ds(base + (EPC // CHUNK) * CHUNK, rem)])
    plsc.subcore_barrier()

    def stage_body(st, carry):
        bufbase = lax.rem(st, 2) * GSTAGE

        @pl.when(st == 0)
        def _():
            r0 = w * NG
            pltpu.async_copy(subj2.at[pl.ds(r0, GSTAGE)],
                             idx_s.at[pl.ds(0, GSTAGE)], sem_i)
            pltpu.async_copy(rel2.at[pl.ds(r0, GSTAGE)],
                             idx_r.at[pl.ds(0, GSTAGE)], sem_i)
            pltpu.async_copy(obj2.at[pl.ds(r0, GSTAGE)],
                             idx_o.at[pl.ds(0, GSTAGE)], sem_i)
        for _ in range(3):
            pltpu.make_async_copy(subj2.at[pl.ds(0, GSTAGE)],
                                  idx_s.at[pl.ds(0, GSTAGE)], sem_i).wait()

        @pl.when(st + 1 < NSTAGE)
        def _():
            r1 = w * NG + (st + 1) * GSTAGE
            nbase = (GSTAGE - bufbase)
            pltpu.async_copy(subj2.at[pl.ds(r1, GSTAGE)],
                             idx_s.at[pl.ds(nbase, GSTAGE)], sem_i)
            pltpu.async_copy(rel2.at[pl.ds(r1, GSTAGE)],
                             idx_r.at[pl.ds(nbase, GSTAGE)], sem_i)
            pltpu.async_copy(obj2.at[pl.ds(r1, GSTAGE)],
                             idx_o.at[pl.ds(nbase, GSTAGE)], sem_i)

        ge, gr, sc = {}, {}, {}
        ge[0] = pltpu.async_copy(eT.at[idx_s.at[bufbase]],
                                 rows_e.at[pl.ds(0, SUB)], sem_e)
        gr[0] = pltpu.async_copy(rd_sh.at[idx_r.at[bufbase]],
                                 rows_r.at[pl.ds(0, SUB)], sem_r)
        for p in range(GSTAGE):
            slot = p % 2
            if p + 1 < GSTAGE:
                if p >= 1:
                    sc[p - 1].wait()
                off = (1 - slot) * SUB
                ge[p + 1] = pltpu.async_copy(
                    eT.at[idx_s.at[bufbase + p + 1]],
                    rows_e.at[pl.ds(off, SUB)], sem_e)
                gr[p + 1] = pltpu.async_copy(
                    rd_sh.at[idx_r.at[bufbase + p + 1]],
                    rows_r.at[pl.ds(off, SUB)], sem_r)
            ge[p].wait()
            gr[p].wait()

            soff = slot * SUB

            @plsc.parallel_loop(0, SUB, 1, unroll=4)
            def _(r):
                rows_e[soff + r, pl.ds(0, 16)] = (
                    rows_e[soff + r, pl.ds(0, 16)] *
                    rows_r[soff + r, pl.ds(0, 16)])
                rows_e[soff + r, pl.ds(16, 16)] = (
                    rows_e[soff + r, pl.ds(16, 16)] *
                    rows_r[soff + r, pl.ds(16, 16)])

            sc[p] = pltpu.async_copy(rows_e.at[pl.ds(soff, SUB)],
                                     acc.at[idx_o.at[bufbase + p]],
                                     sem_w, add=True)
        sc[GSTAGE - 2].wait()
        sc[GSTAGE - 1].wait()
        return carry
    lax.fori_loop(0, NSTAGE, stage_body, 0)

    plsc.subcore_barrier()
    pltpu.sync_copy(acc.at[pl.ds(base, EPC)],
                    out.at[pl.ds(c * E + base, EPC)])


def _follow_sc(eT, rdT, subj2, rel2, obj2):
    f = pl.kernel(
        _follow_body,
        out_type=jax.ShapeDtypeStruct((NC * E, B), jnp.float32),
        mesh=plsc.VectorSubcoreMesh(core_axis_name="c", subcore_axis_name="s"),
        scratch_types=[
            pltpu.VMEM_SHARED((E, B), jnp.float32),
            pltpu.VMEM_SHARED((RPAD, B), jnp.float32),
            pltpu.VMEM((2 * GSTAGE, SUB), jnp.int32),
            pltpu.VMEM((2 * GSTAGE, SUB), jnp.int32),
            pltpu.VMEM((2 * GSTAGE, SUB), jnp.int32),
            pltpu.VMEM((CHUNK, B), jnp.float32),
            pltpu.VMEM((CHUNK, B), jnp.float32),
            pltpu.SemaphoreType.DMA,
            pltpu.SemaphoreType.DMA,
            pltpu.SemaphoreType.DMA,
            pltpu.SemaphoreType.DMA,
        ],
        compiler_params=pltpu.CompilerParams(use_tc_tiling_on_sc=False),
    )
    return f(eT, rdT, subj2, rel2, obj2)


def _prep_body(qs, W0, b0, W1, b1, re_, rdT0_o, rdT1_o, gt_o, hs_o):
    relv = re_[...]
    rows = lax.broadcasted_iota(jnp.int32, (R, B), 0)
    qcol = qs[...][:, 0]
    qm = rows == qcol[None, :]
    q_emb = lax.dot_general(jnp.where(qm, 1.0, 0.0), relv,
                            (((0,), (0,)), ((), ())),
                            preferred_element_type=jnp.float32)

    def rel_dist_T(W, b):
        cq = jnp.tanh(jnp.dot(q_emb, W[...],
                              preferred_element_type=jnp.float32) + b[...])
        lgT = lax.dot_general(relv, cq, (((1,), (1,)), ((), ())),
                              preferred_element_type=jnp.float32)
        mx = jnp.max(lgT, axis=0, keepdims=True)
        ex = jnp.exp(lgT - mx)
        return ex / jnp.sum(ex, axis=0, keepdims=True)

    rdT0 = rel_dist_T(W0, b0)
    rdT1 = rel_dist_T(W1, b1)
    zpad = jnp.zeros((RPAD - R, B), jnp.float32)
    rdT0_o[...] = jnp.concatenate([rdT0, zpad], axis=0)
    rdT1_o[...] = jnp.concatenate([rdT1, zpad], axis=0)

    gt_o[...] = jnp.sum(jnp.where(qm, rdT0, 0.0), axis=0).reshape(1, B)

    def argmax0(rdT):
        amax = jnp.max(rdT, axis=0, keepdims=True)
        return jnp.min(jnp.where(rdT == amax, rows, R), axis=0)

    r0 = argmax0(rdT0)
    r1 = argmax0(rdT1)
    cond2 = (jnp.abs(r0 - r1) == 1) & (jnp.minimum(r0, r1) % 2 == 0)
    nself = (r0 == 0).astype(jnp.int32) + (r1 == 0).astype(jnp.int32)
    cond3 = nself == 1
    hs = (1.0 - cond2.astype(jnp.float32)) * (1.0 - cond3.astype(jnp.float32))
    hs_o[...] = hs.reshape(1, B)


def _prep(questions, W0, b0, W1, b1, rel_emb):
    return pl.pallas_call(
        _prep_body,
        out_shape=[
            jax.ShapeDtypeStruct((RPAD, B), jnp.float32),
            jax.ShapeDtypeStruct((RPAD, B), jnp.float32),
            jax.ShapeDtypeStruct((1, B), jnp.float32),
            jax.ShapeDtypeStruct((1, B), jnp.float32),
        ],
    )(questions, W0, b0, W1, b1, rel_emb)


def _combine0_body(gt, p0, p1, ansT, out):
    x = p0[0] + p1[0] - ansT[...] * gt[...]
    out[...] = jnp.where(x > 1.0, 1.0, x)


def _combine0(gt, P3, ansT):
    return pl.pallas_call(
        _combine0_body,
        grid=(NBLK,),
        in_specs=[
            pl.BlockSpec((1, B), lambda i: (0, 0)),
            pl.BlockSpec((1, BLK, B), lambda i: (0, i, 0)),
            pl.BlockSpec((1, BLK, B), lambda i: (1, i, 0)),
            pl.BlockSpec((BLK, B), lambda i: (i, 0)),
        ],
        out_specs=pl.BlockSpec((BLK, B), lambda i: (i, 0)),
        out_shape=jax.ShapeDtypeStruct((E, B), jnp.float32),
    )(gt, P3, P3, ansT)


def _pass1_body(hs, p0, p1, esT, ansT, emb, UT_o, s_o, q2_o, ea_o):
    i = pl.program_id(0)
    x = p0[0] + p1[0]
    x = jnp.where(x > 1.0, 1.0, x)
    x = x * (1.0 - esT[...] * (1.0 - hs[...]))

    @pl.when(i == 0)
    def _():
        UT_o[...] = jnp.zeros((D, B), jnp.float32)
        s_o[...] = jnp.zeros((1, B), jnp.float32)
        q2_o[...] = jnp.zeros((1, B), jnp.float32)
        ea_o[...] = jnp.zeros((1, B), jnp.float32)

    UT_o[...] += lax.dot_general(emb[...], x, (((0,), (0,)), ((), ())),
                                 preferred_element_type=jnp.float32)
    s_o[...] += jnp.sum(x, axis=0, keepdims=True)
    q2_o[...] += jnp.sum(x * x, axis=0, keepdims=True)
    ea_o[...] += jnp.sum(ansT[...] * x, axis=0, keepdims=True)


def _pass1(hs, P3, esT, ansT, ent_emb):
    return pl.pallas_call(
        _pass1_body,
        grid=(NBLK,),
        in_specs=[
            pl.BlockSpec((1, B), lambda i: (0, 0)),
            pl.BlockSpec((1, BLK, B), lambda i: (0, i, 0)),
            pl.BlockSpec((1, BLK, B), lambda i: (1, i, 0)),
            pl.BlockSpec((BLK, B), lambda i: (i, 0)),
            pl.BlockSpec((BLK, B), lambda i: (i, 0)),
            pl.BlockSpec((BLK, D), lambda i: (i, 0)),
        ],
        out_specs=[
            pl.BlockSpec((D, B), lambda i: (0, 0)),
            pl.BlockSpec((1, B), lambda i: (0, 0)),
            pl.BlockSpec((1, B), lambda i: (0, 0)),
            pl.BlockSpec((1, B), lambda i: (0, 0)),
        ],
        out_shape=[
            jax.ShapeDtypeStruct((D, B), jnp.float32),
            jax.ShapeDtypeStruct((1, B), jnp.float32),
            jax.ShapeDtypeStruct((1, B), jnp.float32),
            jax.ShapeDtypeStruct((1, B), jnp.float32),
        ],
    )(hs, P3, P3, esT, ansT, ent_emb)


def _pass2_body(UT, s, q2, ea, emb, ansT, bias,
                mr_o, sr_o, pa_o, ls_o, lp_o):
    i = pl.program_id(0)
    pmT = UT[...] / (s[...] + 1e-6)
    P = lax.dot_general(emb[...], pmT, (((1,), (0,)), ((), ())),
                        preferred_element_type=jnp.float32) + bias[...]
    bm = jnp.max(P, axis=0, keepdims=True)
    bpa = jnp.sum(ansT[...] * P, axis=0, keepdims=True)

    @pl.when(i == 0)
    def _():
        mr_o[...] = bm
        sr_o[...] = jnp.sum(jnp.exp(P - bm), axis=0, keepdims=True)
        pa_o[...] = bpa

    @pl.when(i > 0)
    def _():
        nm = jnp.maximum(mr_o[...], bm)
        sr_o[...] = (sr_o[...] * jnp.exp(mr_o[...] - nm) +
                     jnp.sum(jnp.exp(P - nm), axis=0, keepdims=True))
        mr_o[...] = nm
        pa_o[...] += bpa

    @pl.when(i == NBLK2 - 1)
    def _():
        lse = jnp.log(sr_o[...]) + mr_o[...]
        lp_o[...] = (-jnp.sum(pa_o[...] - lse) / B).reshape(1, 1)
        eav = ea[...]
        ls_o[...] = ((jnp.sum(q2[...]) - jnp.sum(eav * eav) +
                      jnp.sum(10.0 * (eav - 1.0) ** 2)) / (E * B)).reshape(1, 1)


def _pass2(UT, s, q2, ea, embP, ansTP, biasP):
    return pl.pallas_call(
        _pass2_body,
        grid=(NBLK2,),
        in_specs=[
            pl.BlockSpec((D, B), lambda i: (0, 0)),
            pl.BlockSpec((1, B), lambda i: (0, 0)),
            pl.BlockSpec((1, B), lambda i: (0, 0)),
            pl.BlockSpec((1, B), lambda i: (0, 0)),
            pl.BlockSpec((BLK2, D), lambda i: (i, 0)),
            pl.BlockSpec((BLK2, B), lambda i: (i, 0)),
            pl.BlockSpec((BLK2, 1), lambda i: (i, 0)),
        ],
        out_specs=[
            pl.BlockSpec((1, B), lambda i: (0, 0)),
            pl.BlockSpec((1, B), lambda i: (0, 0)),
            pl.BlockSpec((1, B), lambda i: (0, 0)),
            pl.BlockSpec((1, 1), lambda i: (0, 0)),
            pl.BlockSpec((1, 1), lambda i: (0, 0)),
        ],
        out_shape=[
            jax.ShapeDtypeStruct((1, B), jnp.float32),
            jax.ShapeDtypeStruct((1, B), jnp.float32),
            jax.ShapeDtypeStruct((1, B), jnp.float32),
            jax.ShapeDtypeStruct((1, 1), jnp.float32),
            jax.ShapeDtypeStruct((1, 1), jnp.float32),
        ],
    )(UT, s, q2, ea, embP, ansTP, biasP)


def kernel(questions, e_s, answers, subj_idx, rel_idx, obj_idx,
           W_step0, b_step0, W_step1, b_step1, W_cq, b_cq,
           rel_emb, ent_emb, ent_bias):
    eT0 = e_s.T
    ansT = answers.T
    npad = TP - T
    subj2 = jnp.concatenate(
        [subj_idx, jnp.zeros((npad,), jnp.int32)]).reshape(TP // SUB, SUB)
    rel2 = jnp.concatenate(
        [rel_idx, jnp.full((npad,), R, jnp.int32)]).reshape(TP // SUB, SUB)
    obj2 = jnp.concatenate(
        [obj_idx, jnp.zeros((npad,), jnp.int32)]).reshape(TP // SUB, SUB)

    rdT0, rdT1, gt, hs = _prep(questions, W_step0, b_step0,
                               W_step1, b_step1, rel_emb)

    P0 = _follow_sc(eT0, rdT0, subj2, rel2, obj2).reshape(NC, E, B)
    e1T = _combine0(gt, P0, ansT)
    P1 = _follow_sc(e1T, rdT1, subj2, rel2, obj2).reshape(NC, E, B)

    UT, s, q2, ea = _pass1(hs, P1, eT0, ansT, ent_emb)

    embP = jnp.pad(ent_emb, ((0, EP - E), (0, 0)))
    ansTP = jnp.pad(ansT, ((0, EP - E), (0, 0)))
    biasP = jnp.pad(ent_bias, (0, EP - E),
                    constant_values=-1e30).reshape(EP, 1)
    _, _, _, ls, lp = _pass2(UT, s, q2, ea, embP, ansTP, biasP)
    return (ls[0, 0], lp[0, 0])


# speedup vs baseline: 16.9004x; 1.0100x over previous
import jax
import jax.numpy as jnp
from jax import lax
from jax.experimental import pallas as pl
from jax.experimental.pallas import tpu as pltpu
from jax.experimental.pallas import tpu_sc as plsc

E = 50000
R = 500
D = 128
B = 32
T = 800000

NC, NS = 2, 16
NW = NC * NS
SUB = 128
RPAD = 512
TP = 802816
TW = TP // NW
NG = TW // SUB
GSTAGE = 7
NSTAGE = NG // GSTAGE
ESLOT = 3
CHUNK = ESLOT * SUB
EPC = E // NS

EP = 50176
BLK = 2000
NBLK = E // BLK
BLK2 = EP // 8
NBLK2 = 8


def _follow_body(eT, rdT, subj2, rel2, obj2, out,
                 acc, rd_sh, idx_s, idx_r, idx_o, rows_e, rows_r,
                 sem_e, sem_r, sem_i, sem_w):
    c = lax.axis_index("c")
    s = lax.axis_index("s")
    w = s * NC + c
    base = s * EPC

    @pl.when(s == 0)
    def _():
        pltpu.sync_copy(rdT, rd_sh)

    def zrow(i, carry):
        rows_e[i, pl.ds(0, 16)] = jnp.zeros((16,), jnp.float32)
        rows_e[i, pl.ds(16, 16)] = jnp.zeros((16,), jnp.float32)
        return carry
    lax.fori_loop(0, CHUNK, zrow, 0)
    for k in range(EPC // CHUNK):
        pltpu.sync_copy(rows_e.at[pl.ds(0, CHUNK)],
                        acc.at[pl.ds(base + k * CHUNK, CHUNK)])
    rem = EPC % CHUNK
    if rem:
        pltpu.sync_copy(rows_e.at[pl.ds(0, rem)],
                        acc.at[pl.ds(base + (EPC // CHUNK) * CHUNK, rem)])
    plsc.subcore_barrier()

    def stage_body(st, carry):
        bufbase = lax.rem(st, 2) * GSTAGE

        @pl.when(st == 0)
        def _():
            r0 = w * NG
            pltpu.async_copy(subj2.at[pl.ds(r0, GSTAGE)],
                             idx_s.at[pl.ds(0, GSTAGE)], sem_i)
            pltpu.async_copy(rel2.at[pl.ds(r0, GSTAGE)],
                             idx_r.at[pl.ds(0, GSTAGE)], sem_i)
            pltpu.async_copy(obj2.at[pl.ds(r0, GSTAGE)],
                             idx_o.at[pl.ds(0, GSTAGE)], sem_i)
        for _ in range(3):
            pltpu.make_async_copy(subj2.at[pl.ds(0, GSTAGE)],
                                  idx_s.at[pl.ds(0, GSTAGE)], sem_i).wait()

        @pl.when(st + 1 < NSTAGE)
        def _():
            r1 = w * NG + (st + 1) * GSTAGE
            nbase = (GSTAGE - bufbase)
            pltpu.async_copy(subj2.at[pl.ds(r1, GSTAGE)],
                             idx_s.at[pl.ds(nbase, GSTAGE)], sem_i)
            pltpu.async_copy(rel2.at[pl.ds(r1, GSTAGE)],
                             idx_r.at[pl.ds(nbase, GSTAGE)], sem_i)
            pltpu.async_copy(obj2.at[pl.ds(r1, GSTAGE)],
                             idx_o.at[pl.ds(nbase, GSTAGE)], sem_i)

        ge, gr, sc = {}, {}, {}
        ge[0] = pltpu.async_copy(eT.at[idx_s.at[bufbase]],
                                 rows_e.at[pl.ds(0, SUB)], sem_e)
        ge[1] = pltpu.async_copy(eT.at[idx_s.at[bufbase + 1]],
                                 rows_e.at[pl.ds(SUB, SUB)], sem_e)
        gr[0] = pltpu.async_copy(rd_sh.at[idx_r.at[bufbase]],
                                 rows_r.at[pl.ds(0, SUB)], sem_r)
        for p in range(GSTAGE):
            eoff = (p % ESLOT) * SUB
            roff = (p % 2) * SUB
            if p + 2 < GSTAGE:
                if p >= 1:
                    sc[p - 1].wait()
                ge[p + 2] = pltpu.async_copy(
                    eT.at[idx_s.at[bufbase + p + 2]],
                    rows_e.at[pl.ds(((p + 2) % ESLOT) * SUB, SUB)], sem_e)
            if p + 1 < GSTAGE:
                gr[p + 1] = pltpu.async_copy(
                    rd_sh.at[idx_r.at[bufbase + p + 1]],
                    rows_r.at[pl.ds(((p + 1) % 2) * SUB, SUB)], sem_r)
            ge[p].wait()
            gr[p].wait()

            @plsc.parallel_loop(0, SUB, 1, unroll=4)
            def _(r):
                rows_e[eoff + r, pl.ds(0, 16)] = (
                    rows_e[eoff + r, pl.ds(0, 16)] *
                    rows_r[roff + r, pl.ds(0, 16)])
                rows_e[eoff + r, pl.ds(16, 16)] = (
                    rows_e[eoff + r, pl.ds(16, 16)] *
                    rows_r[roff + r, pl.ds(16, 16)])

            sc[p] = pltpu.async_copy(rows_e.at[pl.ds(eoff, SUB)],
                                     acc.at[idx_o.at[bufbase + p]],
                                     sem_w, add=True)
        for q in range(max(0, GSTAGE - 3), GSTAGE):
            sc[q].wait()
        return carry
    lax.fori_loop(0, NSTAGE, stage_body, 0)

    plsc.subcore_barrier()
    pltpu.sync_copy(acc.at[pl.ds(base, EPC)],
                    out.at[pl.ds(c * E + base, EPC)])


def _follow_sc(eT, rdT, subj2, rel2, obj2):
    f = pl.kernel(
        _follow_body,
        out_type=jax.ShapeDtypeStruct((NC * E, B), jnp.float32),
        mesh=plsc.VectorSubcoreMesh(core_axis_name="c", subcore_axis_name="s"),
        scratch_types=[
            pltpu.VMEM_SHARED((E, B), jnp.float32),
            pltpu.VMEM_SHARED((RPAD, B), jnp.float32),
            pltpu.VMEM((2 * GSTAGE, SUB), jnp.int32),
            pltpu.VMEM((2 * GSTAGE, SUB), jnp.int32),
            pltpu.VMEM((2 * GSTAGE, SUB), jnp.int32),
            pltpu.VMEM((CHUNK, B), jnp.float32),
            pltpu.VMEM((2 * SUB, B), jnp.float32),
            pltpu.SemaphoreType.DMA,
            pltpu.SemaphoreType.DMA,
            pltpu.SemaphoreType.DMA,
            pltpu.SemaphoreType.DMA,
        ],
        compiler_params=pltpu.CompilerParams(use_tc_tiling_on_sc=False),
    )
    return f(eT, rdT, subj2, rel2, obj2)


def _prep_body(qs, W0, b0, W1, b1, re_, rdT0_o, rdT1_o, gt_o, hs_o):
    relv = re_[...]
    rows = lax.broadcasted_iota(jnp.int32, (R, B), 0)
    qcol = qs[...][:, 0]
    qm = rows == qcol[None, :]
    q_emb = lax.dot_general(jnp.where(qm, 1.0, 0.0), relv,
                            (((0,), (0,)), ((), ())),
                            preferred_element_type=jnp.float32)

    def rel_dist_T(W, b):
        cq = jnp.tanh(jnp.dot(q_emb, W[...],
                              preferred_element_type=jnp.float32) + b[...])
        lgT = lax.dot_general(relv, cq, (((1,), (1,)), ((), ())),
                              preferred_element_type=jnp.float32)
        mx = jnp.max(lgT, axis=0, keepdims=True)
        ex = jnp.exp(lgT - mx)
        return ex / jnp.sum(ex, axis=0, keepdims=True)

    rdT0 = rel_dist_T(W0, b0)
    rdT1 = rel_dist_T(W1, b1)
    zpad = jnp.zeros((RPAD - R, B), jnp.float32)
    rdT0_o[...] = jnp.concatenate([rdT0, zpad], axis=0)
    rdT1_o[...] = jnp.concatenate([rdT1, zpad], axis=0)

    gt_o[...] = jnp.sum(jnp.where(qm, rdT0, 0.0), axis=0).reshape(1, B)

    def argmax0(rdT):
        amax = jnp.max(rdT, axis=0, keepdims=True)
        return jnp.min(jnp.where(rdT == amax, rows, R), axis=0)

    r0 = argmax0(rdT0)
    r1 = argmax0(rdT1)
    cond2 = (jnp.abs(r0 - r1) == 1) & (jnp.minimum(r0, r1) % 2 == 0)
    nself = (r0 == 0).astype(jnp.int32) + (r1 == 0).astype(jnp.int32)
    cond3 = nself == 1
    hs = (1.0 - cond2.astype(jnp.float32)) * (1.0 - cond3.astype(jnp.float32))
    hs_o[...] = hs.reshape(1, B)


def _prep(questions, W0, b0, W1, b1, rel_emb):
    return pl.pallas_call(
        _prep_body,
        out_shape=[
            jax.ShapeDtypeStruct((RPAD, B), jnp.float32),
            jax.ShapeDtypeStruct((RPAD, B), jnp.float32),
            jax.ShapeDtypeStruct((1, B), jnp.float32),
            jax.ShapeDtypeStruct((1, B), jnp.float32),
        ],
    )(questions, W0, b0, W1, b1, rel_emb)


def _combine0_body(gt, p0, p1, ansT, out):
    x = p0[0] + p1[0] - ansT[...] * gt[...]
    out[...] = jnp.where(x > 1.0, 1.0, x)


def _combine0(gt, P3, ansT):
    return pl.pallas_call(
        _combine0_body,
        grid=(NBLK,),
        in_specs=[
            pl.BlockSpec((1, B), lambda i: (0, 0)),
            pl.BlockSpec((1, BLK, B), lambda i: (0, i, 0)),
            pl.BlockSpec((1, BLK, B), lambda i: (1, i, 0)),
            pl.BlockSpec((BLK, B), lambda i: (i, 0)),
        ],
        out_specs=pl.BlockSpec((BLK, B), lambda i: (i, 0)),
        out_shape=jax.ShapeDtypeStruct((E, B), jnp.float32),
    )(gt, P3, P3, ansT)


def _pass1_body(hs, p0, p1, esT, ansT, emb, UT_o, s_o, q2_o, ea_o):
    i = pl.program_id(0)
    x = p0[0] + p1[0]
    x = jnp.where(x > 1.0, 1.0, x)
    x = x * (1.0 - esT[...] * (1.0 - hs[...]))

    @pl.when(i == 0)
    def _():
        UT_o[...] = jnp.zeros((D, B), jnp.float32)
        s_o[...] = jnp.zeros((1, B), jnp.float32)
        q2_o[...] = jnp.zeros((1, B), jnp.float32)
        ea_o[...] = jnp.zeros((1, B), jnp.float32)

    UT_o[...] += lax.dot_general(emb[...], x, (((0,), (0,)), ((), ())),
                                 preferred_element_type=jnp.float32)
    s_o[...] += jnp.sum(x, axis=0, keepdims=True)
    q2_o[...] += jnp.sum(x * x, axis=0, keepdims=True)
    ea_o[...] += jnp.sum(ansT[...] * x, axis=0, keepdims=True)


def _pass1(hs, P3, esT, ansT, ent_emb):
    return pl.pallas_call(
        _pass1_body,
        grid=(NBLK,),
        in_specs=[
            pl.BlockSpec((1, B), lambda i: (0, 0)),
            pl.BlockSpec((1, BLK, B), lambda i: (0, i, 0)),
            pl.BlockSpec((1, BLK, B), lambda i: (1, i, 0)),
            pl.BlockSpec((BLK, B), lambda i: (i, 0)),
            pl.BlockSpec((BLK, B), lambda i: (i, 0)),
            pl.BlockSpec((BLK, D), lambda i: (i, 0)),
        ],
        out_specs=[
            pl.BlockSpec((D, B), lambda i: (0, 0)),
            pl.BlockSpec((1, B), lambda i: (0, 0)),
            pl.BlockSpec((1, B), lambda i: (0, 0)),
            pl.BlockSpec((1, B), lambda i: (0, 0)),
        ],
        out_shape=[
            jax.ShapeDtypeStruct((D, B), jnp.float32),
            jax.ShapeDtypeStruct((1, B), jnp.float32),
            jax.ShapeDtypeStruct((1, B), jnp.float32),
            jax.ShapeDtypeStruct((1, B), jnp.float32),
        ],
    )(hs, P3, P3, esT, ansT, ent_emb)


def _pass2_body(UT, s, q2, ea, emb, ansT, bias,
                mr_o, sr_o, pa_o, ls_o, lp_o):
    i = pl.program_id(0)
    pmT = UT[...] / (s[...] + 1e-6)
    P = lax.dot_general(emb[...], pmT, (((1,), (0,)), ((), ())),
                        preferred_element_type=jnp.float32) + bias[...]
    bm = jnp.max(P, axis=0, keepdims=True)
    bpa = jnp.sum(ansT[...] * P, axis=0, keepdims=True)

    @pl.when(i == 0)
    def _():
        mr_o[...] = bm
        sr_o[...] = jnp.sum(jnp.exp(P - bm), axis=0, keepdims=True)
        pa_o[...] = bpa

    @pl.when(i > 0)
    def _():
        nm = jnp.maximum(mr_o[...], bm)
        sr_o[...] = (sr_o[...] * jnp.exp(mr_o[...] - nm) +
                     jnp.sum(jnp.exp(P - nm), axis=0, keepdims=True))
        mr_o[...] = nm
        pa_o[...] += bpa

    @pl.when(i == NBLK2 - 1)
    def _():
        lse = jnp.log(sr_o[...]) + mr_o[...]
        lp_o[...] = (-jnp.sum(pa_o[...] - lse) / B).reshape(1, 1)
        eav = ea[...]
        ls_o[...] = ((jnp.sum(q2[...]) - jnp.sum(eav * eav) +
                      jnp.sum(10.0 * (eav - 1.0) ** 2)) / (E * B)).reshape(1, 1)


def _pass2(UT, s, q2, ea, embP, ansTP, biasP):
    return pl.pallas_call(
        _pass2_body,
        grid=(NBLK2,),
        in_specs=[
            pl.BlockSpec((D, B), lambda i: (0, 0)),
            pl.BlockSpec((1, B), lambda i: (0, 0)),
            pl.BlockSpec((1, B), lambda i: (0, 0)),
            pl.BlockSpec((1, B), lambda i: (0, 0)),
            pl.BlockSpec((BLK2, D), lambda i: (i, 0)),
            pl.BlockSpec((BLK2, B), lambda i: (i, 0)),
            pl.BlockSpec((BLK2, 1), lambda i: (i, 0)),
        ],
        out_specs=[
            pl.BlockSpec((1, B), lambda i: (0, 0)),
            pl.BlockSpec((1, B), lambda i: (0, 0)),
            pl.BlockSpec((1, B), lambda i: (0, 0)),
            pl.BlockSpec((1, 1), lambda i: (0, 0)),
            pl.BlockSpec((1, 1), lambda i: (0, 0)),
        ],
        out_shape=[
            jax.ShapeDtypeStruct((1, B), jnp.float32),
            jax.ShapeDtypeStruct((1, B), jnp.float32),
            jax.ShapeDtypeStruct((1, B), jnp.float32),
            jax.ShapeDtypeStruct((1, 1), jnp.float32),
            jax.ShapeDtypeStruct((1, 1), jnp.float32),
        ],
    )(UT, s, q2, ea, embP, ansTP, biasP)


def kernel(questions, e_s, answers, subj_idx, rel_idx, obj_idx,
           W_step0, b_step0, W_step1, b_step1, W_cq, b_cq,
           rel_emb, ent_emb, ent_bias):
    eT0 = e_s.T
    ansT = answers.T
    npad = TP - T
    subj2 = jnp.concatenate(
        [subj_idx, jnp.zeros((npad,), jnp.int32)]).reshape(TP // SUB, SUB)
    rel2 = jnp.concatenate(
        [rel_idx, jnp.full((npad,), R, jnp.int32)]).reshape(TP // SUB, SUB)
    obj2 = jnp.concatenate(
        [obj_idx, jnp.zeros((npad,), jnp.int32)]).reshape(TP // SUB, SUB)

    rdT0, rdT1, gt, hs = _prep(questions, W_step0, b_step0,
                               W_step1, b_step1, rel_emb)

    P0 = _follow_sc(eT0, rdT0, subj2, rel2, obj2).reshape(NC, E, B)
    e1T = _combine0(gt, P0, ansT)
    P1 = _follow_sc(e1T, rdT1, subj2, rel2, obj2).reshape(NC, E, B)

    UT, s, q2, ea = _pass1(hs, P1, eT0, ansT, ent_emb)

    embP = jnp.pad(ent_emb, ((0, EP - E), (0, 0)))
    ansTP = jnp.pad(ansT, ((0, EP - E), (0, 0)))
    biasP = jnp.pad(ent_bias, (0, EP - E),
                    constant_values=-1e30).reshape(EP, 1)
    _, _, _, ls, lp = _pass2(UT, s, q2, ea, embP, ansTP, biasP)
    return (ls[0, 0], lp[0, 0])


# speedup vs baseline: 17.4956x; 1.0352x over previous
import jax
import jax.numpy as jnp
from jax import lax
from jax.experimental import pallas as pl
from jax.experimental.pallas import tpu as pltpu
from jax.experimental.pallas import tpu_sc as plsc

E = 50000
R = 500
D = 128
B = 32
T = 800000

NC, NS = 2, 16
NW = NC * NS
SUB = 128
RPAD = 512
TP = 802816
TW = TP // NW
NG = TW // SUB
GSTAGE = 7
NSTAGE = NG // GSTAGE
ESLOT = 3
CHUNK = ESLOT * SUB
EPC = E // NS

ORD = [v for i in range(16) for v in (i, i + 16)]
INV = [ORD.index(i) for i in range(32)]

EP = 50176
BLK = 2000
NBLK = E // BLK
BLK2 = EP // 8
NBLK2 = 8


def _follow_body(eT, rdT, subj2, rel2, obj2, out,
                 acc, rd_sh, idx_s, idx_r, idx_o, rows_e, rows_r, rows_p,
                 sem_e, sem_r, sem_i, sem_w):
    c = lax.axis_index("c")
    s = lax.axis_index("s")
    w = s * NC + c
    base = s * EPC

    @pl.when(s == 0)
    def _():
        pltpu.sync_copy(rdT, rd_sh)

    PCH = 2 * SUB
    def zrow(i, carry):
        rows_p[i, pl.ds(0, 16)] = jnp.zeros((16,), jnp.float32)
        rows_p[i, pl.ds(16, 16)] = jnp.zeros((16,), jnp.float32)
        return carry
    lax.fori_loop(0, PCH, zrow, 0)
    for k in range(EPC // PCH):
        pltpu.sync_copy(rows_p.at[pl.ds(0, PCH)],
                        acc.at[pl.ds(base + k * PCH, PCH)])
    rem = EPC % PCH
    if rem:
        pltpu.sync_copy(rows_p.at[pl.ds(0, rem)],
                        acc.at[pl.ds(base + (EPC // PCH) * PCH, rem)])
    plsc.subcore_barrier()

    def stage_body(st, carry):
        bufbase = lax.rem(st, 2) * GSTAGE

        @pl.when(st == 0)
        def _():
            r0 = w * NG
            pltpu.async_copy(subj2.at[pl.ds(r0, GSTAGE)],
                             idx_s.at[pl.ds(0, GSTAGE)], sem_i)
            pltpu.async_copy(rel2.at[pl.ds(r0, GSTAGE)],
                             idx_r.at[pl.ds(0, GSTAGE)], sem_i)
            pltpu.async_copy(obj2.at[pl.ds(r0, GSTAGE)],
                             idx_o.at[pl.ds(0, GSTAGE)], sem_i)
        for _ in range(3):
            pltpu.make_async_copy(subj2.at[pl.ds(0, GSTAGE)],
                                  idx_s.at[pl.ds(0, GSTAGE)], sem_i).wait()

        @pl.when(st + 1 < NSTAGE)
        def _():
            r1 = w * NG + (st + 1) * GSTAGE
            nbase = (GSTAGE - bufbase)
            pltpu.async_copy(subj2.at[pl.ds(r1, GSTAGE)],
                             idx_s.at[pl.ds(nbase, GSTAGE)], sem_i)
            pltpu.async_copy(rel2.at[pl.ds(r1, GSTAGE)],
                             idx_r.at[pl.ds(nbase, GSTAGE)], sem_i)
            pltpu.async_copy(obj2.at[pl.ds(r1, GSTAGE)],
                             idx_o.at[pl.ds(nbase, GSTAGE)], sem_i)

        ge, gr, sc = {}, {}, {}
        ge[0] = pltpu.async_copy(eT.at[idx_s.at[bufbase]],
                                 rows_e.at[pl.ds(0, SUB)], sem_e)
        ge[1] = pltpu.async_copy(eT.at[idx_s.at[bufbase + 1]],
                                 rows_e.at[pl.ds(SUB, SUB)], sem_e)
        gr[0] = pltpu.async_copy(rd_sh.at[idx_r.at[bufbase]],
                                 rows_r.at[pl.ds(0, SUB)], sem_r)
        for p in range(GSTAGE):
            eoff = (p % ESLOT) * SUB
            roff = (p % 2) * SUB
            if p + 2 < GSTAGE:
                if p >= 1:
                    sc[p - 1].wait()
                ge[p + 2] = pltpu.async_copy(
                    eT.at[idx_s.at[bufbase + p + 2]],
                    rows_e.at[pl.ds(((p + 2) % ESLOT) * SUB, SUB)], sem_e)
            if p + 1 < GSTAGE:
                gr[p + 1] = pltpu.async_copy(
                    rd_sh.at[idx_r.at[bufbase + p + 1]],
                    rows_r.at[pl.ds(((p + 1) % 2) * SUB, SUB)], sem_r)
            poff = (p % 2) * SUB
            ge[p].wait()
            gr[p].wait()

            @plsc.parallel_loop(0, SUB, 1, unroll=4)
            def _(r):
                a, b = plsc.unpack(rows_e[eoff + r, :],
                                   format=plsc.PackFormat.INTERLEAVED)
                rows_p[poff + r, pl.ds(0, 16)] = (
                    a * rows_r[roff + r, pl.ds(0, 16)])
                rows_p[poff + r, pl.ds(16, 16)] = (
                    b * rows_r[roff + r, pl.ds(16, 16)])

            sc[p] = pltpu.async_copy(rows_p.at[pl.ds(poff, SUB)],
                                     acc.at[idx_o.at[bufbase + p]],
                                     sem_w, add=True)
        for q in range(max(0, GSTAGE - 3), GSTAGE):
            sc[q].wait()
        return carry
    lax.fori_loop(0, NSTAGE, stage_body, 0)

    plsc.subcore_barrier()
    pltpu.sync_copy(acc.at[pl.ds(base, EPC)],
                    out.at[pl.ds(c * E + base, EPC)])


def _follow_sc(eT, rdT, subj2, rel2, obj2):
    f = pl.kernel(
        _follow_body,
        out_type=jax.ShapeDtypeStruct((NC * E, B), jnp.float32),
        mesh=plsc.VectorSubcoreMesh(core_axis_name="c", subcore_axis_name="s"),
        scratch_types=[
            pltpu.VMEM_SHARED((E, B), jnp.float32),
            pltpu.VMEM_SHARED((RPAD, B), jnp.float32),
            pltpu.VMEM((2 * GSTAGE, SUB), jnp.int32),
            pltpu.VMEM((2 * GSTAGE, SUB), jnp.int32),
            pltpu.VMEM((2 * GSTAGE, SUB), jnp.int32),
            pltpu.VMEM((CHUNK, B), jnp.bfloat16),
            pltpu.VMEM((2 * SUB, B), jnp.float32),
            pltpu.VMEM((2 * SUB, B), jnp.float32),
            pltpu.SemaphoreType.DMA,
            pltpu.SemaphoreType.DMA,
            pltpu.SemaphoreType.DMA,
            pltpu.SemaphoreType.DMA,
        ],
        compiler_params=pltpu.CompilerParams(use_tc_tiling_on_sc=False,
                                             needs_layout_passes=False),
    )
    return f(eT, rdT, subj2, rel2, obj2)


def _prep_body(qs, W0, b0, W1, b1, re_, rdT0_o, rdT1_o, gt_o, hs_o):
    relv = re_[...]
    rows = lax.broadcasted_iota(jnp.int32, (R, B), 0)
    qcol = qs[...][:, 0]
    qm = rows == qcol[None, :]
    q_emb = lax.dot_general(jnp.where(qm, 1.0, 0.0), relv,
                            (((0,), (0,)), ((), ())),
                            preferred_element_type=jnp.float32)

    def rel_dist_T(W, b):
        cq = jnp.tanh(jnp.dot(q_emb, W[...],
                              preferred_element_type=jnp.float32) + b[...])
        lgT = lax.dot_general(relv, cq, (((1,), (1,)), ((), ())),
                              preferred_element_type=jnp.float32)
        mx = jnp.max(lgT, axis=0, keepdims=True)
        ex = jnp.exp(lgT - mx)
        return ex / jnp.sum(ex, axis=0, keepdims=True)

    rdT0 = rel_dist_T(W0, b0)
    rdT1 = rel_dist_T(W1, b1)
    zpad = jnp.zeros((RPAD - R, B), jnp.float32)
    rdT0_o[...] = jnp.concatenate([rdT0, zpad], axis=0)
    rdT1_o[...] = jnp.concatenate([rdT1, zpad], axis=0)

    gt_o[...] = jnp.sum(jnp.where(qm, rdT0, 0.0), axis=0).reshape(1, B)

    def argmax0(rdT):
        amax = jnp.max(rdT, axis=0, keepdims=True)
        return jnp.min(jnp.where(rdT == amax, rows, R), axis=0)

    r0 = argmax0(rdT0)
    r1 = argmax0(rdT1)
    cond2 = (jnp.abs(r0 - r1) == 1) & (jnp.minimum(r0, r1) % 2 == 0)
    nself = (r0 == 0).astype(jnp.int32) + (r1 == 0).astype(jnp.int32)
    cond3 = nself == 1
    hs = (1.0 - cond2.astype(jnp.float32)) * (1.0 - cond3.astype(jnp.float32))
    hs_o[...] = hs.reshape(1, B)


def _prep(questions, W0, b0, W1, b1, rel_emb):
    return pl.pallas_call(
        _prep_body,
        out_shape=[
            jax.ShapeDtypeStruct((RPAD, B), jnp.float32),
            jax.ShapeDtypeStruct((RPAD, B), jnp.float32),
            jax.ShapeDtypeStruct((1, B), jnp.float32),
            jax.ShapeDtypeStruct((1, B), jnp.float32),
        ],
    )(questions, W0, b0, W1, b1, rel_emb)


def _combine0_body(gt, p0, p1, ansT, out):
    x = p0[0] + p1[0] - ansT[...] * gt[...]
    out[...] = jnp.where(x > 1.0, 1.0, x).astype(jnp.bfloat16)


def _combine0(gt, P3, ansT):
    return pl.pallas_call(
        _combine0_body,
        grid=(NBLK,),
        in_specs=[
            pl.BlockSpec((1, B), lambda i: (0, 0)),
            pl.BlockSpec((1, BLK, B), lambda i: (0, i, 0)),
            pl.BlockSpec((1, BLK, B), lambda i: (1, i, 0)),
            pl.BlockSpec((BLK, B), lambda i: (i, 0)),
        ],
        out_specs=pl.BlockSpec((BLK, B), lambda i: (i, 0)),
        out_shape=jax.ShapeDtypeStruct((E, B), jnp.bfloat16),
    )(gt, P3, P3, ansT)


def _pass1_body(hs, p0, p1, esT, ansT, emb, UT_o, s_o, q2_o, ea_o):
    i = pl.program_id(0)
    x = p0[0] + p1[0]
    x = jnp.where(x > 1.0, 1.0, x)
    x = x * (1.0 - esT[...] * (1.0 - hs[...]))

    @pl.when(i == 0)
    def _():
        UT_o[...] = jnp.zeros((D, B), jnp.float32)
        s_o[...] = jnp.zeros((1, B), jnp.float32)
        q2_o[...] = jnp.zeros((1, B), jnp.float32)
        ea_o[...] = jnp.zeros((1, B), jnp.float32)

    UT_o[...] += lax.dot_general(emb[...], x, (((0,), (0,)), ((), ())),
                                 preferred_element_type=jnp.float32)
    s_o[...] += jnp.sum(x, axis=0, keepdims=True)
    q2_o[...] += jnp.sum(x * x, axis=0, keepdims=True)
    ea_o[...] += jnp.sum(ansT[...] * x, axis=0, keepdims=True)


def _pass1(hs, P3, esT, ansT, ent_emb):
    return pl.pallas_call(
        _pass1_body,
        grid=(NBLK,),
        in_specs=[
            pl.BlockSpec((1, B), lambda i: (0, 0)),
            pl.BlockSpec((1, BLK, B), lambda i: (0, i, 0)),
            pl.BlockSpec((1, BLK, B), lambda i: (1, i, 0)),
            pl.BlockSpec((BLK, B), lambda i: (i, 0)),
            pl.BlockSpec((BLK, B), lambda i: (i, 0)),
            pl.BlockSpec((BLK, D), lambda i: (i, 0)),
        ],
        out_specs=[
            pl.BlockSpec((D, B), lambda i: (0, 0)),
            pl.BlockSpec((1, B), lambda i: (0, 0)),
            pl.BlockSpec((1, B), lambda i: (0, 0)),
            pl.BlockSpec((1, B), lambda i: (0, 0)),
        ],
        out_shape=[
            jax.ShapeDtypeStruct((D, B), jnp.float32),
            jax.ShapeDtypeStruct((1, B), jnp.float32),
            jax.ShapeDtypeStruct((1, B), jnp.float32),
            jax.ShapeDtypeStruct((1, B), jnp.float32),
        ],
    )(hs, P3, P3, esT, ansT, ent_emb)


def _pass2_body(UT, s, q2, ea, emb, ansT, bias,
                mr_o, sr_o, pa_o, ls_o, lp_o):
    i = pl.program_id(0)
    pmT = UT[...] / (s[...] + 1e-6)
    P = lax.dot_general(emb[...], pmT, (((1,), (0,)), ((), ())),
                        preferred_element_type=jnp.float32) + bias[...]
    bm = jnp.max(P, axis=0, keepdims=True)
    bpa = jnp.sum(ansT[...] * P, axis=0, keepdims=True)

    @pl.when(i == 0)
    def _():
        mr_o[...] = bm
        sr_o[...] = jnp.sum(jnp.exp(P - bm), axis=0, keepdims=True)
        pa_o[...] = bpa

    @pl.when(i > 0)
    def _():
        nm = jnp.maximum(mr_o[...], bm)
        sr_o[...] = (sr_o[...] * jnp.exp(mr_o[...] - nm) +
                     jnp.sum(jnp.exp(P - nm), axis=0, keepdims=True))
        mr_o[...] = nm
        pa_o[...] += bpa

    @pl.when(i == NBLK2 - 1)
    def _():
        lse = jnp.log(sr_o[...]) + mr_o[...]
        lp_o[...] = (-jnp.sum(pa_o[...] - lse) / B).reshape(1, 1)
        eav = ea[...]
        ls_o[...] = ((jnp.sum(q2[...]) - jnp.sum(eav * eav) +
                      jnp.sum(10.0 * (eav - 1.0) ** 2)) / (E * B)).reshape(1, 1)


def _pass2(UT, s, q2, ea, embP, ansTP, biasP):
    return pl.pallas_call(
        _pass2_body,
        grid=(NBLK2,),
        in_specs=[
            pl.BlockSpec((D, B), lambda i: (0, 0)),
            pl.BlockSpec((1, B), lambda i: (0, 0)),
            pl.BlockSpec((1, B), lambda i: (0, 0)),
            pl.BlockSpec((1, B), lambda i: (0, 0)),
            pl.BlockSpec((BLK2, D), lambda i: (i, 0)),
            pl.BlockSpec((BLK2, B), lambda i: (i, 0)),
            pl.BlockSpec((BLK2, 1), lambda i: (i, 0)),
        ],
        out_specs=[
            pl.BlockSpec((1, B), lambda i: (0, 0)),
            pl.BlockSpec((1, B), lambda i: (0, 0)),
            pl.BlockSpec((1, B), lambda i: (0, 0)),
            pl.BlockSpec((1, 1), lambda i: (0, 0)),
            pl.BlockSpec((1, 1), lambda i: (0, 0)),
        ],
        out_shape=[
            jax.ShapeDtypeStruct((1, B), jnp.float32),
            jax.ShapeDtypeStruct((1, B), jnp.float32),
            jax.ShapeDtypeStruct((1, B), jnp.float32),
            jax.ShapeDtypeStruct((1, 1), jnp.float32),
            jax.ShapeDtypeStruct((1, 1), jnp.float32),
        ],
    )(UT, s, q2, ea, embP, ansTP, biasP)


def kernel(questions, e_s, answers, subj_idx, rel_idx, obj_idx,
           W_step0, b_step0, W_step1, b_step1, W_cq, b_cq,
           rel_emb, ent_emb, ent_bias):
    ordv = jnp.array(ORD, dtype=jnp.int32)
    invv = jnp.array(INV, dtype=jnp.int32)
    eT0 = e_s.T[:, ordv].astype(jnp.bfloat16)
    esTp = e_s.T[:, ordv]
    ansT = answers.T[:, ordv]
    npad = TP - T
    subj2 = jnp.concatenate(
        [subj_idx, jnp.zeros((npad,), jnp.int32)]).reshape(TP // SUB, SUB)
    rel2 = jnp.concatenate(
        [rel_idx, jnp.full((npad,), R, jnp.int32)]).reshape(TP // SUB, SUB)
    obj2 = jnp.concatenate(
        [obj_idx, jnp.zeros((npad,), jnp.int32)]).reshape(TP // SUB, SUB)

    rdT0, rdT1, gt, hs = _prep(questions, W_step0, b_step0,
                               W_step1, b_step1, rel_emb)
    rdT0 = rdT0[:, ordv]
    rdT1 = rdT1[:, ordv]
    gt = gt[:, ordv]
    hs = hs[:, ordv]

    P0 = _follow_sc(eT0, rdT0, subj2, rel2, obj2).reshape(NC, E, B)
    e1T = _combine0(gt, P0, ansT)
    P1 = _follow_sc(e1T, rdT1, subj2, rel2, obj2).reshape(NC, E, B)

    UT, s, q2, ea = _pass1(hs, P1, esTp, ansT, ent_emb)
    UT = UT[:, invv]
    s = s[:, invv]
    q2 = q2[:, invv]
    ea = ea[:, invv]

    embP = jnp.pad(ent_emb, ((0, EP - E), (0, 0)))
    ansTP = jnp.pad(ansT, ((0, EP - E), (0, 0)))
    biasP = jnp.pad(ent_bias, (0, EP - E),
                    constant_values=-1e30).reshape(EP, 1)
    _, _, _, ls, lp = _pass2(UT, s, q2, ea, embP, ansTP, biasP)
    return (ls[0, 0], lp[0, 0])


# speedup vs baseline: 19.2901x; 1.1026x over previous
import jax
import jax.numpy as jnp
from jax import lax
from jax.experimental import pallas as pl
from jax.experimental.pallas import tpu as pltpu
from jax.experimental.pallas import tpu_sc as plsc

E = 50000
R = 500
D = 128
B = 32
T = 800000

NC, NS = 2, 16
NW = NC * NS
SUB = 128
RPAD = 512
TP = 802816
TW = TP // NW
NG = TW // SUB
GSTAGE = 7
NSTAGE = NG // GSTAGE
ESLOT = 3
CHUNK = ESLOT * SUB
EPC = E // NS

ORD = [v for i in range(16) for v in (i, i + 16)]
INV = [ORD.index(i) for i in range(32)]

EP = 50176
BLK = 2000
NBLK = E // BLK
BLK2 = EP // 8
NBLK2 = 8


def _follow_body(eT, rdT, subj2, rel2, obj2, out,
                 acc, rd_sh, idx_s, idx_r, idx_o, rows_e, rows_r, rows_p,
                 sem_e, sem_r, sem_i, sem_w):
    c = lax.axis_index("c")
    s = lax.axis_index("s")
    w = s * NC + c
    base = s * EPC

    @pl.when(s == 0)
    def _():
        pltpu.sync_copy(rdT, rd_sh)

    PCH = 2 * SUB
    def zrow(i, carry):
        rows_p[i, pl.ds(0, 16)] = jnp.zeros((16,), jnp.float32)
        rows_p[i, pl.ds(16, 16)] = jnp.zeros((16,), jnp.float32)
        return carry
    lax.fori_loop(0, PCH, zrow, 0)
    for k in range(EPC // PCH):
        pltpu.sync_copy(rows_p.at[pl.ds(0, PCH)],
                        acc.at[pl.ds(base + k * PCH, PCH)])
    rem = EPC % PCH
    if rem:
        pltpu.sync_copy(rows_p.at[pl.ds(0, rem)],
                        acc.at[pl.ds(base + (EPC // PCH) * PCH, rem)])
    plsc.subcore_barrier()

    def stage_body(st, carry):
        bufbase = lax.rem(st, 2) * GSTAGE

        @pl.when(st == 0)
        def _():
            r0 = w * NG
            pltpu.async_copy(subj2.at[pl.ds(r0, GSTAGE)],
                             idx_s.at[pl.ds(0, GSTAGE)], sem_i)
            pltpu.async_copy(rel2.at[pl.ds(r0, GSTAGE)],
                             idx_r.at[pl.ds(0, GSTAGE)], sem_i)
            pltpu.async_copy(obj2.at[pl.ds(r0, GSTAGE)],
                             idx_o.at[pl.ds(0, GSTAGE)], sem_i)
        for _ in range(3):
            pltpu.make_async_copy(subj2.at[pl.ds(0, GSTAGE)],
                                  idx_s.at[pl.ds(0, GSTAGE)], sem_i).wait()

        @pl.when(st + 1 < NSTAGE)
        def _():
            r1 = w * NG + (st + 1) * GSTAGE
            nbase = (GSTAGE - bufbase)
            pltpu.async_copy(subj2.at[pl.ds(r1, GSTAGE)],
                             idx_s.at[pl.ds(nbase, GSTAGE)], sem_i)
            pltpu.async_copy(rel2.at[pl.ds(r1, GSTAGE)],
                             idx_r.at[pl.ds(nbase, GSTAGE)], sem_i)
            pltpu.async_copy(obj2.at[pl.ds(r1, GSTAGE)],
                             idx_o.at[pl.ds(nbase, GSTAGE)], sem_i)

        ge, gr, sc = {}, {}, {}
        ge[0] = pltpu.async_copy(eT.at[idx_s.at[bufbase]],
                                 rows_e.at[pl.ds(0, SUB)], sem_e)
        ge[1] = pltpu.async_copy(eT.at[idx_s.at[bufbase + 1]],
                                 rows_e.at[pl.ds(SUB, SUB)], sem_e)
        gr[0] = pltpu.async_copy(rd_sh.at[idx_r.at[bufbase]],
                                 rows_r.at[pl.ds(0, SUB)], sem_r)
        for p in range(GSTAGE):
            eoff = (p % ESLOT) * SUB
            roff = (p % 2) * SUB
            if p + 2 < GSTAGE:
                if p >= 1:
                    sc[p - 1].wait()
                ge[p + 2] = pltpu.async_copy(
                    eT.at[idx_s.at[bufbase + p + 2]],
                    rows_e.at[pl.ds(((p + 2) % ESLOT) * SUB, SUB)], sem_e)
            if p + 1 < GSTAGE:
                gr[p + 1] = pltpu.async_copy(
                    rd_sh.at[idx_r.at[bufbase + p + 1]],
                    rows_r.at[pl.ds(((p + 1) % 2) * SUB, SUB)], sem_r)
            poff = (p % 2) * SUB
            ge[p].wait()
            gr[p].wait()

            @plsc.parallel_loop(0, SUB, 1, unroll=4)
            def _(r):
                a, b = plsc.unpack(rows_e[eoff + r, :],
                                   format=plsc.PackFormat.INTERLEAVED)
                rows_p[poff + r, pl.ds(0, 16)] = (
                    a * rows_r[roff + r, pl.ds(0, 16)])
                rows_p[poff + r, pl.ds(16, 16)] = (
                    b * rows_r[roff + r, pl.ds(16, 16)])

            sc[p] = pltpu.async_copy(rows_p.at[pl.ds(poff, SUB)],
                                     acc.at[idx_o.at[bufbase + p]],
                                     sem_w, add=True)
        for q in range(max(0, GSTAGE - 3), GSTAGE):
            sc[q].wait()
        return carry
    lax.fori_loop(0, NSTAGE, stage_body, 0)

    plsc.subcore_barrier()
    pltpu.sync_copy(acc.at[pl.ds(base, EPC)],
                    out.at[pl.ds(c * E + base, EPC)])


def _follow0_body(eT, rdT, subj2, rel2, obj2, hm2, out,
                  acc, rd_sh, idx_s, idx_r, idx_o, mbuf,
                  rows_e, rows_r, rows_p, sem_e, sem_r, sem_i, sem_w):
    c = lax.axis_index("c")
    s = lax.axis_index("s")
    w = s * NC + c
    base = s * EPC

    @pl.when(s == 0)
    def _():
        pltpu.sync_copy(rdT, rd_sh)

    PCH = 2 * SUB
    def zrow(i, carry):
        rows_p[i, pl.ds(0, 16)] = jnp.zeros((16,), jnp.float32)
        rows_p[i, pl.ds(16, 16)] = jnp.zeros((16,), jnp.float32)
        return carry
    lax.fori_loop(0, PCH, zrow, 0)
    for k in range(EPC // PCH):
        pltpu.sync_copy(rows_p.at[pl.ds(0, PCH)],
                        acc.at[pl.ds(base + k * PCH, PCH)])
    rem = EPC % PCH
    if rem:
        pltpu.sync_copy(rows_p.at[pl.ds(0, rem)],
                        acc.at[pl.ds(base + (EPC // PCH) * PCH, rem)])
    plsc.subcore_barrier()

    def stage_body(st, carry):
        bufbase = lax.rem(st, 2) * GSTAGE

        @pl.when(st == 0)
        def _():
            r0 = w * NG
            pltpu.async_copy(subj2.at[pl.ds(r0, GSTAGE)],
                             idx_s.at[pl.ds(0, GSTAGE)], sem_i)
            pltpu.async_copy(rel2.at[pl.ds(r0, GSTAGE)],
                             idx_r.at[pl.ds(0, GSTAGE)], sem_i)
            pltpu.async_copy(obj2.at[pl.ds(r0, GSTAGE)],
                             idx_o.at[pl.ds(0, GSTAGE)], sem_i)
            pltpu.async_copy(hm2.at[pl.ds(r0, GSTAGE)],
                             mbuf.at[pl.ds(0, GSTAGE)], sem_i)
        for _ in range(4):
            pltpu.make_async_copy(subj2.at[pl.ds(0, GSTAGE)],
                                  idx_s.at[pl.ds(0, GSTAGE)], sem_i).wait()

        @pl.when(st + 1 < NSTAGE)
        def _():
            r1 = w * NG + (st + 1) * GSTAGE
            nbase = (GSTAGE - bufbase)
            pltpu.async_copy(subj2.at[pl.ds(r1, GSTAGE)],
                             idx_s.at[pl.ds(nbase, GSTAGE)], sem_i)
            pltpu.async_copy(rel2.at[pl.ds(r1, GSTAGE)],
                             idx_r.at[pl.ds(nbase, GSTAGE)], sem_i)
            pltpu.async_copy(obj2.at[pl.ds(r1, GSTAGE)],
                             idx_o.at[pl.ds(nbase, GSTAGE)], sem_i)
            pltpu.async_copy(hm2.at[pl.ds(r1, GSTAGE)],
                             mbuf.at[pl.ds(nbase, GSTAGE)], sem_i)

        for p in range(GSTAGE):
            row = bufbase + p
            m = mbuf[row, pl.ds(0, 16)]
            for k in range(1, 8):
                m = m + mbuf[row, pl.ds(16 * k, 16)]
            tot = jnp.sum(m, axis=0)

            @pl.when(tot > 0.0)
            def _():
                cpe = pltpu.async_copy(eT.at[idx_s.at[row]],
                                       rows_e.at[pl.ds(0, SUB)], sem_e)
                cpr = pltpu.async_copy(rd_sh.at[idx_r.at[row]],
                                       rows_r.at[pl.ds(0, SUB)], sem_r)
                cpe.wait()
                cpr.wait()

                @plsc.parallel_loop(0, SUB, 1, unroll=4)
                def _(r):
                    a, b = plsc.unpack(rows_e[r, :],
                                       format=plsc.PackFormat.INTERLEAVED)
                    rows_p[r, pl.ds(0, 16)] = a * rows_r[r, pl.ds(0, 16)]
                    rows_p[r, pl.ds(16, 16)] = b * rows_r[r, pl.ds(16, 16)]

                pltpu.sync_copy(rows_p.at[pl.ds(0, SUB)],
                                acc.at[idx_o.at[row]], add=True)
        return carry
    lax.fori_loop(0, NSTAGE, stage_body, 0)

    plsc.subcore_barrier()
    pltpu.sync_copy(acc.at[pl.ds(base, EPC)],
                    out.at[pl.ds(c * E + base, EPC)])


def _follow0_sc(eT, rdT, subj2, rel2, obj2, hm2):
    f = pl.kernel(
        _follow0_body,
        out_type=jax.ShapeDtypeStruct((NC * E, B), jnp.float32),
        mesh=plsc.VectorSubcoreMesh(core_axis_name="c", subcore_axis_name="s"),
        scratch_types=[
            pltpu.VMEM_SHARED((E, B), jnp.float32),
            pltpu.VMEM_SHARED((RPAD, B), jnp.float32),
            pltpu.VMEM((2 * GSTAGE, SUB), jnp.int32),
            pltpu.VMEM((2 * GSTAGE, SUB), jnp.int32),
            pltpu.VMEM((2 * GSTAGE, SUB), jnp.int32),
            pltpu.VMEM((2 * GSTAGE, SUB), jnp.float32),
            pltpu.VMEM((CHUNK, B), jnp.bfloat16),
            pltpu.VMEM((2 * SUB, B), jnp.float32),
            pltpu.VMEM((2 * SUB, B), jnp.float32),
            pltpu.SemaphoreType.DMA,
            pltpu.SemaphoreType.DMA,
            pltpu.SemaphoreType.DMA,
            pltpu.SemaphoreType.DMA,
        ],
        compiler_params=pltpu.CompilerParams(use_tc_tiling_on_sc=False,
                                             needs_layout_passes=False),
    )
    return f(eT, rdT, subj2, rel2, obj2, hm2)


def _follow_sc(eT, rdT, subj2, rel2, obj2):
    f = pl.kernel(
        _follow_body,
        out_type=jax.ShapeDtypeStruct((NC * E, B), jnp.float32),
        mesh=plsc.VectorSubcoreMesh(core_axis_name="c", subcore_axis_name="s"),
        scratch_types=[
            pltpu.VMEM_SHARED((E, B), jnp.float32),
            pltpu.VMEM_SHARED((RPAD, B), jnp.float32),
            pltpu.VMEM((2 * GSTAGE, SUB), jnp.int32),
            pltpu.VMEM((2 * GSTAGE, SUB), jnp.int32),
            pltpu.VMEM((2 * GSTAGE, SUB), jnp.int32),
            pltpu.VMEM((CHUNK, B), jnp.bfloat16),
            pltpu.VMEM((2 * SUB, B), jnp.float32),
            pltpu.VMEM((2 * SUB, B), jnp.float32),
            pltpu.SemaphoreType.DMA,
            pltpu.SemaphoreType.DMA,
            pltpu.SemaphoreType.DMA,
            pltpu.SemaphoreType.DMA,
        ],
        compiler_params=pltpu.CompilerParams(use_tc_tiling_on_sc=False,
                                             needs_layout_passes=False),
    )
    return f(eT, rdT, subj2, rel2, obj2)


def _prep_body(qs, W0, b0, W1, b1, re_, rdT0_o, rdT1_o, gt_o, hs_o):
    relv = re_[...]
    rows = lax.broadcasted_iota(jnp.int32, (R, B), 0)
    qcol = qs[...][:, 0]
    qm = rows == qcol[None, :]
    q_emb = lax.dot_general(jnp.where(qm, 1.0, 0.0), relv,
                            (((0,), (0,)), ((), ())),
                            preferred_element_type=jnp.float32)

    def rel_dist_T(W, b):
        cq = jnp.tanh(jnp.dot(q_emb, W[...],
                              preferred_element_type=jnp.float32) + b[...])
        lgT = lax.dot_general(relv, cq, (((1,), (1,)), ((), ())),
                              preferred_element_type=jnp.float32)
        mx = jnp.max(lgT, axis=0, keepdims=True)
        ex = jnp.exp(lgT - mx)
        return ex / jnp.sum(ex, axis=0, keepdims=True)

    rdT0 = rel_dist_T(W0, b0)
    rdT1 = rel_dist_T(W1, b1)
    zpad = jnp.zeros((RPAD - R, B), jnp.float32)
    rdT0_o[...] = jnp.concatenate([rdT0, zpad], axis=0)
    rdT1_o[...] = jnp.concatenate([rdT1, zpad], axis=0)

    gt_o[...] = jnp.sum(jnp.where(qm, rdT0, 0.0), axis=0).reshape(1, B)

    def argmax0(rdT):
        amax = jnp.max(rdT, axis=0, keepdims=True)
        return jnp.min(jnp.where(rdT == amax, rows, R), axis=0)

    r0 = argmax0(rdT0)
    r1 = argmax0(rdT1)
    cond2 = (jnp.abs(r0 - r1) == 1) & (jnp.minimum(r0, r1) % 2 == 0)
    nself = (r0 == 0).astype(jnp.int32) + (r1 == 0).astype(jnp.int32)
    cond3 = nself == 1
    hs = (1.0 - cond2.astype(jnp.float32)) * (1.0 - cond3.astype(jnp.float32))
    hs_o[...] = hs.reshape(1, B)


def _prep(questions, W0, b0, W1, b1, rel_emb):
    return pl.pallas_call(
        _prep_body,
        out_shape=[
            jax.ShapeDtypeStruct((RPAD, B), jnp.float32),
            jax.ShapeDtypeStruct((RPAD, B), jnp.float32),
            jax.ShapeDtypeStruct((1, B), jnp.float32),
            jax.ShapeDtypeStruct((1, B), jnp.float32),
        ],
    )(questions, W0, b0, W1, b1, rel_emb)


def _heads_body(esT, h_o):
    i = pl.program_id(0)

    @pl.when(i == 0)
    def _():
        h_o[...] = jnp.zeros((1, B), jnp.float32)

    rows = (jnp.float32(i * BLK) +
            lax.broadcasted_iota(jnp.int32, (BLK, B), 0).astype(jnp.float32))
    h_o[...] += jnp.sum(rows * esT[...], axis=0, keepdims=True)


def _heads(esT):
    return pl.pallas_call(
        _heads_body,
        grid=(NBLK,),
        in_specs=[pl.BlockSpec((BLK, B), lambda i: (i, 0))],
        out_specs=pl.BlockSpec((1, B), lambda i: (0, 0)),
        out_shape=jax.ShapeDtypeStruct((1, B), jnp.float32),
    )(esT)


HBLK = (TP // SUB) // 8


def _hitmask_body(h, subj, out):
    sb = subj[...]
    m = jnp.zeros((HBLK, SUB), jnp.int32)
    for b in range(B):
        hb = h[0, b].astype(jnp.int32)
        m = m | (sb == hb).astype(jnp.int32)
    out[...] = m.astype(jnp.float32)


def _hitmask(h, subj2):
    return pl.pallas_call(
        _hitmask_body,
        grid=(8,),
        in_specs=[
            pl.BlockSpec((1, B), lambda i: (0, 0)),
            pl.BlockSpec((HBLK, SUB), lambda i: (i, 0)),
        ],
        out_specs=pl.BlockSpec((HBLK, SUB), lambda i: (i, 0)),
        out_shape=jax.ShapeDtypeStruct((TP // SUB, SUB), jnp.float32),
    )(h, subj2)


def _combine0_body(gt, p0, p1, ansT, out):
    x = p0[0] + p1[0] - ansT[...] * gt[...]
    out[...] = jnp.where(x > 1.0, 1.0, x).astype(jnp.bfloat16)


def _combine0(gt, P3, ansT):
    return pl.pallas_call(
        _combine0_body,
        grid=(NBLK,),
        in_specs=[
            pl.BlockSpec((1, B), lambda i: (0, 0)),
            pl.BlockSpec((1, BLK, B), lambda i: (0, i, 0)),
            pl.BlockSpec((1, BLK, B), lambda i: (1, i, 0)),
            pl.BlockSpec((BLK, B), lambda i: (i, 0)),
        ],
        out_specs=pl.BlockSpec((BLK, B), lambda i: (i, 0)),
        out_shape=jax.ShapeDtypeStruct((E, B), jnp.bfloat16),
    )(gt, P3, P3, ansT)


def _pass1_body(hs, p0, p1, esT, ansT, emb, UT_o, s_o, q2_o, ea_o):
    i = pl.program_id(0)
    x = p0[0] + p1[0]
    x = jnp.where(x > 1.0, 1.0, x)
    x = x * (1.0 - esT[...] * (1.0 - hs[...]))

    @pl.when(i == 0)
    def _():
        UT_o[...] = jnp.zeros((D, B), jnp.float32)
        s_o[...] = jnp.zeros((1, B), jnp.float32)
        q2_o[...] = jnp.zeros((1, B), jnp.float32)
        ea_o[...] = jnp.zeros((1, B), jnp.float32)

    UT_o[...] += lax.dot_general(emb[...], x, (((0,), (0,)), ((), ())),
                                 preferred_element_type=jnp.float32)
    s_o[...] += jnp.sum(x, axis=0, keepdims=True)
    q2_o[...] += jnp.sum(x * x, axis=0, keepdims=True)
    ea_o[...] += jnp.sum(ansT[...] * x, axis=0, keepdims=True)


def _pass1(hs, P3, esT, ansT, ent_emb):
    return pl.pallas_call(
        _pass1_body,
        grid=(NBLK,),
        in_specs=[
            pl.BlockSpec((1, B), lambda i: (0, 0)),
            pl.BlockSpec((1, BLK, B), lambda i: (0, i, 0)),
            pl.BlockSpec((1, BLK, B), lambda i: (1, i, 0)),
            pl.BlockSpec((BLK, B), lambda i: (i, 0)),
            pl.BlockSpec((BLK, B), lambda i: (i, 0)),
            pl.BlockSpec((BLK, D), lambda i: (i, 0)),
        ],
        out_specs=[
            pl.BlockSpec((D, B), lambda i: (0, 0)),
            pl.BlockSpec((1, B), lambda i: (0, 0)),
            pl.BlockSpec((1, B), lambda i: (0, 0)),
            pl.BlockSpec((1, B), lambda i: (0, 0)),
        ],
        out_shape=[
            jax.ShapeDtypeStruct((D, B), jnp.float32),
            jax.ShapeDtypeStruct((1, B), jnp.float32),
            jax.ShapeDtypeStruct((1, B), jnp.float32),
            jax.ShapeDtypeStruct((1, B), jnp.float32),
        ],
    )(hs, P3, P3, esT, ansT, ent_emb)


def _pass2_body(UT, s, q2, ea, emb, ansT, bias,
                mr_o, sr_o, pa_o, ls_o, lp_o):
    i = pl.program_id(0)
    pmT = UT[...] / (s[...] + 1e-6)
    P = lax.dot_general(emb[...], pmT, (((1,), (0,)), ((), ())),
                        preferred_element_type=jnp.float32) + bias[...]
    bm = jnp.max(P, axis=0, keepdims=True)
    bpa = jnp.sum(ansT[...] * P, axis=0, keepdims=True)

    @pl.when(i == 0)
    def _():
        mr_o[...] = bm
        sr_o[...] = jnp.sum(jnp.exp(P - bm), axis=0, keepdims=True)
        pa_o[...] = bpa

    @pl.when(i > 0)
    def _():
        nm = jnp.maximum(mr_o[...], bm)
        sr_o[...] = (sr_o[...] * jnp.exp(mr_o[...] - nm) +
                     jnp.sum(jnp.exp(P - nm), axis=0, keepdims=True))
        mr_o[...] = nm
        pa_o[...] += bpa

    @pl.when(i == NBLK2 - 1)
    def _():
        lse = jnp.log(sr_o[...]) + mr_o[...]
        lp_o[...] = (-jnp.sum(pa_o[...] - lse) / B).reshape(1, 1)
        eav = ea[...]
        ls_o[...] = ((jnp.sum(q2[...]) - jnp.sum(eav * eav) +
                      jnp.sum(10.0 * (eav - 1.0) ** 2)) / (E * B)).reshape(1, 1)


def _pass2(UT, s, q2, ea, embP, ansTP, biasP):
    return pl.pallas_call(
        _pass2_body,
        grid=(NBLK2,),
        in_specs=[
            pl.BlockSpec((D, B), lambda i: (0, 0)),
            pl.BlockSpec((1, B), lambda i: (0, 0)),
            pl.BlockSpec((1, B), lambda i: (0, 0)),
            pl.BlockSpec((1, B), lambda i: (0, 0)),
            pl.BlockSpec((BLK2, D), lambda i: (i, 0)),
            pl.BlockSpec((BLK2, B), lambda i: (i, 0)),
            pl.BlockSpec((BLK2, 1), lambda i: (i, 0)),
        ],
        out_specs=[
            pl.BlockSpec((1, B), lambda i: (0, 0)),
            pl.BlockSpec((1, B), lambda i: (0, 0)),
            pl.BlockSpec((1, B), lambda i: (0, 0)),
            pl.BlockSpec((1, 1), lambda i: (0, 0)),
            pl.BlockSpec((1, 1), lambda i: (0, 0)),
        ],
        out_shape=[
            jax.ShapeDtypeStruct((1, B), jnp.float32),
            jax.ShapeDtypeStruct((1, B), jnp.float32),
            jax.ShapeDtypeStruct((1, B), jnp.float32),
            jax.ShapeDtypeStruct((1, 1), jnp.float32),
            jax.ShapeDtypeStruct((1, 1), jnp.float32),
        ],
    )(UT, s, q2, ea, embP, ansTP, biasP)


def kernel(questions, e_s, answers, subj_idx, rel_idx, obj_idx,
           W_step0, b_step0, W_step1, b_step1, W_cq, b_cq,
           rel_emb, ent_emb, ent_bias):
    ordv = jnp.array(ORD, dtype=jnp.int32)
    invv = jnp.array(INV, dtype=jnp.int32)
    eT0 = e_s.T[:, ordv].astype(jnp.bfloat16)
    esTp = e_s.T[:, ordv]
    ansT = answers.T[:, ordv]
    npad = TP - T
    subj2 = jnp.concatenate(
        [subj_idx, jnp.zeros((npad,), jnp.int32)]).reshape(TP // SUB, SUB)
    rel2 = jnp.concatenate(
        [rel_idx, jnp.full((npad,), R, jnp.int32)]).reshape(TP // SUB, SUB)
    obj2 = jnp.concatenate(
        [obj_idx, jnp.zeros((npad,), jnp.int32)]).reshape(TP // SUB, SUB)

    rdT0, rdT1, gt, hs = _prep(questions, W_step0, b_step0,
                               W_step1, b_step1, rel_emb)
    rdT0 = rdT0[:, ordv]
    rdT1 = rdT1[:, ordv]
    gt = gt[:, ordv]
    hs = hs[:, ordv]

    hm2 = _hitmask(_heads(esTp), subj2)
    P0 = _follow0_sc(eT0, rdT0, subj2, rel2, obj2, hm2).reshape(NC, E, B)
    e1T = _combine0(gt, P0, ansT)
    P1 = _follow_sc(e1T, rdT1, subj2, rel2, obj2).reshape(NC, E, B)

    UT, s, q2, ea = _pass1(hs, P1, esTp, ansT, ent_emb)
    UT = UT[:, invv]
    s = s[:, invv]
    q2 = q2[:, invv]
    ea = ea[:, invv]

    embP = jnp.pad(ent_emb, ((0, EP - E), (0, 0)))
    ansTP = jnp.pad(ansT, ((0, EP - E), (0, 0)))
    biasP = jnp.pad(ent_bias, (0, EP - E),
                    constant_values=-1e30).reshape(EP, 1)
    _, _, _, ls, lp = _pass2(UT, s, q2, ea, embP, ansTP, biasP)
    return (ls[0, 0], lp[0, 0])
